# Initial kernel scaffold; baseline (speedup 1.0000x reference)
import functools

import jax
import jax.numpy as jnp
from jax import lax
from jax.experimental import pallas as pl
from jax.experimental.pallas import tpu as pltpu
from jax.experimental.pallas import tpu_sc as plsc

N_NODES = 100000
N_EDGES = 1600000
HIDDEN = 32
HALF = 16
N_GRAPHS = 128
NC = 2
NS = 16
L = 16

_MESH = plsc.VectorSubcoreMesh(
    core_axis_name="c", subcore_axis_name="s", num_cores=NC, num_subcores=NS)


def _zero_vmem(ref, n):
  zeros = jnp.zeros((L,), ref.dtype)

  def body(i, _):
    ref[pl.ds(i * L, L)] = zeros
    return 0

  lax.fori_loop(0, n // L, body, 0)


_KA = 2000
_EPW_A = N_EDGES // (NC * NS)
_NCH_A = _EPW_A // _KA
_ZCH_A = N_NODES // _KA


def _sca_body(x_hbm, src_hbm, dst_hbm, w_hbm, out, acc, srcv, dstv, wv, xg,
              zbuf, sem):
  c = lax.axis_index("c")
  s = lax.axis_index("s")

  _zero_vmem(zbuf, _KA)
  for j in range(-(-_ZCH_A // NS)):
    cid = s + NS * j

    @pl.when(cid < _ZCH_A)
    def _():
      pltpu.sync_copy(zbuf, acc.at[pl.ds(cid * _KA, _KA)])

  plsc.subcore_barrier()

  base0 = (c * NS + s) * _EPW_A

  def chunk(i, _):
    base = base0 + i * _KA
    pltpu.sync_copy(src_hbm.at[pl.ds(base, _KA)], srcv)
    pltpu.sync_copy(dst_hbm.at[pl.ds(base, _KA)], dstv)
    pltpu.sync_copy(w_hbm.at[pl.ds(base, _KA)], wv)
    pltpu.async_copy(x_hbm.at[srcv], xg, sem).wait()

    def scale(j, _):
      sl = pl.ds(j * L, L)
      xg[sl] = xg[sl] * wv[sl]
      return 0

    lax.fori_loop(0, _KA // L, scale, 0)
    pltpu.sync_copy(xg, acc.at[dstv], add=True)
    return 0

  lax.fori_loop(0, _NCH_A, chunk, 0)
  plsc.subcore_barrier()

  for j in range(-(-_ZCH_A // NS)):
    cid = s + NS * j

    @pl.when(cid < _ZCH_A)
    def _():
      sl = pl.ds(cid * _KA, _KA)
      pltpu.sync_copy(acc.at[sl], out.at[c, sl])


@jax.jit
def _sc_a(x1d, src, dst, w):
  return pl.kernel(
      _sca_body,
      out_type=jax.ShapeDtypeStruct((NC, N_NODES), jnp.float32),
      mesh=_MESH,
      scratch_types=[
          pltpu.VMEM_SHARED((N_NODES,), jnp.float32),
          pltpu.VMEM((_KA,), jnp.int32),
          pltpu.VMEM((_KA,), jnp.int32),
          pltpu.VMEM((_KA,), jnp.float32),
          pltpu.VMEM((_KA,), jnp.float32),
          pltpu.VMEM((_KA,), jnp.float32),
          pltpu.SemaphoreType.DMA,
      ],
  )(x1d, src, dst, w)


_KB = 800
_EPT_B = N_EDGES // NS
_NCH_B = _EPT_B // _KB
_ZCH_B = N_NODES // _KB


def _scb_body(h1a_hbm, h1b_hbm, src_hbm, dst_hbm, w_hbm, out0, out1,
              acc, srcv, dstv, wv, rows, sem):
  c = lax.axis_index("c")
  s = lax.axis_index("s")

  def zrow(i, _):
    rows[i, :] = jnp.zeros((L,), jnp.float32)
    return 0

  lax.fori_loop(0, _KB, zrow, 0)

  for j in range(-(-_ZCH_B // NS)):
    cid = s + NS * j

    @pl.when(cid < _ZCH_B)
    def _():
      pltpu.sync_copy(rows, acc.at[pl.ds(cid * _KB, _KB)])

  plsc.subcore_barrier()

  def chunk(i, _):
    base = s * _EPT_B + i * _KB
    pltpu.sync_copy(src_hbm.at[pl.ds(base, _KB)], srcv)
    pltpu.sync_copy(dst_hbm.at[pl.ds(base, _KB)], dstv)
    pltpu.sync_copy(w_hbm.at[pl.ds(base, _KB)], wv)

    @pl.when(c == 0)
    def _():
      pltpu.async_copy(h1a_hbm.at[srcv], rows, sem).wait()

    @pl.when(c == 1)
    def _():
      pltpu.async_copy(h1b_hbm.at[srcv], rows, sem).wait()

    def scale(e, _):
      rows[e, :] = rows[e, :] * wv[e]
      return 0

    lax.fori_loop(0, _KB, scale, 0)
    pltpu.sync_copy(rows, acc.at[dstv], add=True)
    return 0

  lax.fori_loop(0, _NCH_B, chunk, 0)
  plsc.subcore_barrier()

  for j in range(-(-_ZCH_B // NS)):
    cid = s + NS * j

    @pl.when(cid < _ZCH_B)
    def _():
      sl = pl.ds(cid * _KB, _KB)

      @pl.when(c == 0)
      def _():
        pltpu.sync_copy(acc.at[sl], out0.at[sl])

      @pl.when(c == 1)
      def _():
        pltpu.sync_copy(acc.at[sl], out1.at[sl])


@jax.jit
def _sc_b(h1a, h1b, src, dst, w):
  return pl.kernel(
      _scb_body,
      out_type=(
          jax.ShapeDtypeStruct((N_NODES, HALF), jnp.float32),
          jax.ShapeDtypeStruct((N_NODES, HALF), jnp.float32),
      ),
      mesh=_MESH,
      scratch_types=[
          pltpu.VMEM_SHARED((N_NODES, HALF), jnp.float32),
          pltpu.VMEM((_KB,), jnp.int32),
          pltpu.VMEM((_KB,), jnp.int32),
          pltpu.VMEM((_KB,), jnp.float32),
          pltpu.VMEM((_KB, HALF), jnp.float32),
          pltpu.SemaphoreType.DMA,
      ],
  )(h1a, h1b, src, dst, w)


_KC = 2000
_EPW_C = N_EDGES // (NC * NS)
_NCH_C = _EPW_C // _KC
_NCH_N = N_NODES // _KC
_ACC_SZ = L * N_GRAPHS


def _scc_body(z0_hbm, z1_hbm, r0_hbm, r1_hbm, batch_hbm, src_hbm, dst_hbm,
              w_hbm, out, srcv, dstv, wv, g0, g1, bdg,
              a0, a1, a2, a3, a4, obuf, sem):
  c = lax.axis_index("c")
  s = lax.axis_index("s")
  wid = c * NS + s
  lane128 = lax.iota(jnp.int32, L) * N_GRAPHS
  ones = jnp.full((L,), 1.0, jnp.float32)

  for a in (a0, a1, a2, a3, a4):
    _zero_vmem(a, _ACC_SZ)

  def echunk(i, _):
    base = wid * _EPW_C + i * _KC
    pltpu.sync_copy(src_hbm.at[pl.ds(base, _KC)], srcv)
    pltpu.sync_copy(dst_hbm.at[pl.ds(base, _KC)], dstv)
    pltpu.sync_copy(w_hbm.at[pl.ds(base, _KC)], wv)
    pltpu.async_copy(z0_hbm.at[srcv], g0, sem).wait()
    pltpu.async_copy(z1_hbm.at[srcv], g1, sem).wait()
    pltpu.async_copy(batch_hbm.at[dstv], bdg, sem).wait()

    def vec(j, _):
      sl = pl.ds(j * L, L)
      idx = bdg[sl] + lane128
      w16 = wv[sl]
      plsc.addupdate_scatter(a0, [idx], g0[sl] * w16)
      plsc.addupdate_scatter(a1, [idx], g1[sl] * w16)
      return 0

    lax.fori_loop(0, _KC // L, vec, 0)
    return 0

  lax.fori_loop(0, _NCH_C, echunk, 0)

  for j in range(-(-_NCH_N // (NC * NS))):
    cid = wid + NC * NS * j

    @pl.when(cid < _NCH_N)
    def _():
      sl_h = pl.ds(cid * _KC, _KC)
      pltpu.sync_copy(batch_hbm.at[sl_h], bdg)
      pltpu.sync_copy(r0_hbm.at[sl_h], g0)
      pltpu.sync_copy(r1_hbm.at[sl_h], g1)

      def vec(j2, _):
        sl = pl.ds(j2 * L, L)
        idx = bdg[sl] + lane128
        plsc.addupdate_scatter(a2, [idx], g0[sl])
        plsc.addupdate_scatter(a3, [idx], g1[sl])
        plsc.addupdate_scatter(a4, [idx], ones)
        return 0

      lax.fori_loop(0, _KC // L, vec, 0)

  for p, a in enumerate((a0, a1, a2, a3, a4)):
    for j in range(N_GRAPHS // L):
      v = jnp.zeros((L,), jnp.float32)
      for lane in range(L):
        v = v + a[pl.ds(lane * N_GRAPHS + j * L, L)]
      obuf[p, pl.ds(j * L, L)] = v

  pltpu.sync_copy(obuf, out.at[wid])


@jax.jit
def _sc_c(z0, z1, r0, r1, batch, src, dst, w):
  return pl.kernel(
      _scc_body,
      out_type=jax.ShapeDtypeStruct((NC * NS, 5, N_GRAPHS), jnp.float32),
      mesh=_MESH,
      scratch_types=[
          pltpu.VMEM((_KC,), jnp.int32),
          pltpu.VMEM((_KC,), jnp.int32),
          pltpu.VMEM((_KC,), jnp.float32),
          pltpu.VMEM((_KC,), jnp.float32),
          pltpu.VMEM((_KC,), jnp.float32),
          pltpu.VMEM((_KC,), jnp.int32),
          pltpu.VMEM((_ACC_SZ,), jnp.float32),
          pltpu.VMEM((_ACC_SZ,), jnp.float32),
          pltpu.VMEM((_ACC_SZ,), jnp.float32),
          pltpu.VMEM((_ACC_SZ,), jnp.float32),
          pltpu.VMEM((_ACC_SZ,), jnp.float32),
          pltpu.VMEM((5, N_GRAPHS), jnp.float32),
          pltpu.SemaphoreType.DMA,
      ],
  )(z0, z1, r0, r1, batch, src, dst, w)


_BT = 5000
_GT = N_NODES // _BT


def _tc1_body(ap_ref, x_ref, wrel_ref, wroot_ref, b1_ref, ha_ref, hb_ref):
  a = ap_ref[0, :] + ap_ref[1, :]
  xv = x_ref[:, 0]
  pre = (a[:, None] * wrel_ref[0, :][None, :]
         + xv[:, None] * wroot_ref[0, :][None, :] + b1_ref[0, :][None, :])
  h = jnp.maximum(pre, 0.0)
  ha_ref[...] = h[:, :HALF]
  hb_ref[...] = h[:, HALF:]


@jax.jit
def _tc_1(aparts, x, W1_rel, W1_root, b1):
  return pl.pallas_call(
      _tc1_body,
      grid=(_GT,),
      in_specs=[
          pl.BlockSpec((NC, _BT), lambda i: (0, i)),
          pl.BlockSpec((_BT, 1), lambda i: (i, 0)),
          pl.BlockSpec((1, HIDDEN), lambda i: (0, 0)),
          pl.BlockSpec((1, HIDDEN), lambda i: (0, 0)),
          pl.BlockSpec((1, HIDDEN), lambda i: (0, 0)),
      ],
      out_specs=[
          pl.BlockSpec((_BT, HALF), lambda i: (i, 0)),
          pl.BlockSpec((_BT, HALF), lambda i: (i, 0)),
      ],
      out_shape=[
          jax.ShapeDtypeStruct((N_NODES, HALF), jnp.float32),
          jax.ShapeDtypeStruct((N_NODES, HALF), jnp.float32),
      ],
  )(aparts, x, W1_rel, W1_root, b1)


def _tc2_body(g0_ref, g1_ref, ha_ref, hb_ref, w2rel_ref, w2root_ref, b2_ref,
              w3rel_ref, w3root_ref, linw_ref, z0_ref, z1_ref, r0_ref, r1_ref):
  agg = jnp.concatenate([g0_ref[...], g1_ref[...]], axis=1)
  h1 = jnp.concatenate([ha_ref[...], hb_ref[...]], axis=1)
  pre = (jnp.dot(agg, w2rel_ref[...], preferred_element_type=jnp.float32)
         + jnp.dot(h1, w2root_ref[...], preferred_element_type=jnp.float32)
         + b2_ref[0, :][None, :])
  h2 = jnp.maximum(pre, 0.0)
  wz = jnp.dot(w3rel_ref[...], linw_ref[...],
               preferred_element_type=jnp.float32)
  wr = jnp.dot(w3root_ref[...], linw_ref[...],
               preferred_element_type=jnp.float32)
  z = jnp.dot(h2, wz, preferred_element_type=jnp.float32)
  r = jnp.dot(h2, wr, preferred_element_type=jnp.float32)
  z0_ref[...] = z[:, 0:1]
  z1_ref[...] = z[:, 1:2]
  r0_ref[...] = r[:, 0:1]
  r1_ref[...] = r[:, 1:2]


@jax.jit
def _tc_2(g0, g1, ha, hb, W2_rel, W2_root, b2, W3_rel, W3_root, lin_W):
  full = lambda r, c: pl.BlockSpec((r, c), lambda i: (0, 0))
  return pl.pallas_call(
      _tc2_body,
      grid=(_GT,),
      in_specs=[
          pl.BlockSpec((_BT, HALF), lambda i: (i, 0)),
          pl.BlockSpec((_BT, HALF), lambda i: (i, 0)),
          pl.BlockSpec((_BT, HALF), lambda i: (i, 0)),
          pl.BlockSpec((_BT, HALF), lambda i: (i, 0)),
          full(HIDDEN, HIDDEN),
          full(HIDDEN, HIDDEN),
          full(1, HIDDEN),
          full(HIDDEN, HIDDEN),
          full(HIDDEN, HIDDEN),
          full(HIDDEN, 2),
      ],
      out_specs=[pl.BlockSpec((_BT, 1), lambda i: (i, 0))] * 4,
      out_shape=[jax.ShapeDtypeStruct((N_NODES, 1), jnp.float32)] * 4,
  )(g0, g1, ha, hb, W2_rel, W2_root, b2, W3_rel, W3_root, lin_W)


def _tc3_body(p_ref, b3_ref, linw_ref, linb_ref, out_ref):
  sums = jnp.sum(p_ref[...], axis=0)
  e0, e1, r0, r1, n = (sums[0], sums[1], sums[2], sums[3], sums[4])
  cnt = jnp.maximum(n, 1.0)
  base = jnp.dot(b3_ref[...], linw_ref[...],
                 preferred_element_type=jnp.float32)
  nz = (n > 0.0).astype(jnp.float32)
  col0 = (e0 + r0) / cnt + nz * base[0, 0] + linb_ref[0, 0]
  col1 = (e1 + r1) / cnt + nz * base[0, 1] + linb_ref[0, 1]
  out_ref[...] = jnp.stack([col0, col1], axis=1)


@jax.jit
def _tc_3(partials, b3, lin_W, lin_b):
  return pl.pallas_call(
      _tc3_body,
      in_specs=[
          pl.BlockSpec((NC * NS, 5, N_GRAPHS), lambda: (0, 0, 0)),
          pl.BlockSpec((1, HIDDEN), lambda: (0, 0)),
          pl.BlockSpec((HIDDEN, 2), lambda: (0, 0)),
          pl.BlockSpec((1, 2), lambda: (0, 0)),
      ],
      out_specs=pl.BlockSpec((N_GRAPHS, 2), lambda: (0, 0)),
      out_shape=jax.ShapeDtypeStruct((N_GRAPHS, 2), jnp.float32),
  )(partials, b3, lin_W, lin_b)


def kernel(x, edge_index, batch, edge_weight, W1_rel, b1, W1_root, W2_rel, b2,
           W2_root, W3_rel, b3, W3_root, lin_W, lin_b):
  src = edge_index[0]
  dst = edge_index[1]
  x1d = x[:, 0]

  aparts = _sc_a(x1d, src, dst, edge_weight)
  ha, hb = _tc_1(aparts, x, W1_rel, W1_root, b1.reshape(1, HIDDEN))
  g0, g1 = _sc_b(ha, hb, src, dst, edge_weight)
  z0, z1, r0, r1 = _tc_2(g0, g1, ha, hb, W2_rel, W2_root,
                         b2.reshape(1, HIDDEN), W3_rel, W3_root, lin_W)
  partials = _sc_c(z0[:, 0], z1[:, 0], r0[:, 0], r1[:, 0], batch, src, dst,
                   edge_weight)
  return _tc_3(partials, b3.reshape(1, HIDDEN), lin_W, lin_b.reshape(1, 2))


# speedup vs baseline: 13.9575x; 13.9575x over previous
import functools

import jax
import jax.numpy as jnp
from jax import lax
from jax.experimental import pallas as pl
from jax.experimental.pallas import tpu as pltpu
from jax.experimental.pallas import tpu_sc as plsc

N_NODES = 100000
N_EDGES = 1600000
HIDDEN = 32
HALF = 16
N_GRAPHS = 128
NC = 2
NS = 16
L = 16

_MESH = plsc.VectorSubcoreMesh(
    core_axis_name="c", subcore_axis_name="s", num_cores=NC, num_subcores=NS)


def _zero_vmem(ref, n):
  zeros = jnp.zeros((L,), ref.dtype)

  def body(i, _):
    ref[pl.ds(i * L, L)] = zeros
    return 0

  lax.fori_loop(0, n // L, body, 0)


_KA = 2000
_EPW_A = N_EDGES // (NC * NS)
_NCH_A = _EPW_A // _KA
_ZCH_A = N_NODES // _KA


def _sca_body(x_hbm, src_hbm, dst_hbm, w_hbm, out0, out1, acc, srcv, dstv,
              wv, xg, zbuf, sem):
  c = lax.axis_index("c")
  s = lax.axis_index("s")

  _zero_vmem(zbuf, _KA)
  for j in range(-(-_ZCH_A // NS)):
    cid = s + NS * j

    @pl.when(cid < _ZCH_A)
    def _():
      pltpu.sync_copy(zbuf, acc.at[pl.ds(cid * _KA, _KA)])

  plsc.subcore_barrier()

  base0 = (c * NS + s) * _EPW_A

  def chunk(i, _):
    base = base0 + i * _KA
    pltpu.sync_copy(src_hbm.at[pl.ds(base, _KA)], srcv)
    pltpu.sync_copy(dst_hbm.at[pl.ds(base, _KA)], dstv)
    pltpu.sync_copy(w_hbm.at[pl.ds(base, _KA)], wv)
    pltpu.async_copy(x_hbm.at[srcv], xg, sem).wait()

    def scale(j, _):
      sl = pl.ds(j * L, L)
      xg[sl] = xg[sl] * wv[sl]
      return 0

    lax.fori_loop(0, _KA // L, scale, 0)
    pltpu.sync_copy(xg, acc.at[dstv], add=True)
    return 0

  lax.fori_loop(0, _NCH_A, chunk, 0)
  plsc.subcore_barrier()

  for j in range(-(-_ZCH_A // NS)):
    cid = s + NS * j

    @pl.when(cid < _ZCH_A)
    def _():
      sl = pl.ds(cid * _KA, _KA)
      pltpu.sync_copy(acc.at[sl], xg)

      @pl.when(c == 0)
      def _():
        pltpu.sync_copy(xg, out0.at[sl])

      @pl.when(c == 1)
      def _():
        pltpu.sync_copy(xg, out1.at[sl])


@jax.jit
def _sc_a(x1d, src, dst, w):
  return pl.kernel(
      _sca_body,
      out_type=(
          jax.ShapeDtypeStruct((N_NODES,), jnp.float32),
          jax.ShapeDtypeStruct((N_NODES,), jnp.float32),
      ),
      mesh=_MESH,
      scratch_types=[
          pltpu.VMEM_SHARED((N_NODES,), jnp.float32),
          pltpu.VMEM((_KA,), jnp.int32),
          pltpu.VMEM((_KA,), jnp.int32),
          pltpu.VMEM((_KA,), jnp.float32),
          pltpu.VMEM((_KA,), jnp.float32),
          pltpu.VMEM((_KA,), jnp.float32),
          pltpu.SemaphoreType.DMA,
      ],
  )(x1d, src, dst, w)


_KB = 800
_EPT_B = N_EDGES // NS
_NCH_B = _EPT_B // _KB
_ZCH_B = N_NODES // _KB


def _scb_body(h1a_hbm, h1b_hbm, src_hbm, dst_hbm, w_hbm, out0, out1,
              acc, srcv, dstv, wv, rows, sem):
  c = lax.axis_index("c")
  s = lax.axis_index("s")

  def zrow(i, _):
    rows[i, :] = jnp.zeros((L,), jnp.float32)
    return 0

  lax.fori_loop(0, _KB, zrow, 0)

  for j in range(-(-_ZCH_B // NS)):
    cid = s + NS * j

    @pl.when(cid < _ZCH_B)
    def _():
      pltpu.sync_copy(rows, acc.at[pl.ds(cid * _KB, _KB)])

  plsc.subcore_barrier()

  def chunk(i, _):
    base = s * _EPT_B + i * _KB
    pltpu.sync_copy(src_hbm.at[pl.ds(base, _KB)], srcv)
    pltpu.sync_copy(dst_hbm.at[pl.ds(base, _KB)], dstv)
    pltpu.sync_copy(w_hbm.at[pl.ds(base, _KB)], wv.at[pl.ds(0, _KB)])

    @pl.when(c == 0)
    def _():
      pltpu.async_copy(h1a_hbm.at[srcv], rows, sem).wait()

    @pl.when(c == 1)
    def _():
      pltpu.async_copy(h1b_hbm.at[srcv], rows, sem).wait()

    def scale(e, _):
      wvec = wv[pl.ds(e, L)]
      rows[e, :] = rows[e, :] * wvec[0]
      return 0

    lax.fori_loop(0, _KB, scale, 0)
    pltpu.sync_copy(rows, acc.at[dstv], add=True)
    return 0

  lax.fori_loop(0, _NCH_B, chunk, 0)
  plsc.subcore_barrier()

  for j in range(-(-_ZCH_B // NS)):
    cid = s + NS * j

    @pl.when(cid < _ZCH_B)
    def _():
      sl = pl.ds(cid * _KB, _KB)
      pltpu.sync_copy(acc.at[sl], rows)

      @pl.when(c == 0)
      def _():
        pltpu.sync_copy(rows, out0.at[sl])

      @pl.when(c == 1)
      def _():
        pltpu.sync_copy(rows, out1.at[sl])


@jax.jit
def _sc_b(h1a, h1b, src, dst, w):
  return pl.kernel(
      _scb_body,
      out_type=(
          jax.ShapeDtypeStruct((N_NODES, HALF), jnp.float32),
          jax.ShapeDtypeStruct((N_NODES, HALF), jnp.float32),
      ),
      mesh=_MESH,
      compiler_params=pltpu.CompilerParams(use_tc_tiling_on_sc=False),
      scratch_types=[
          pltpu.VMEM_SHARED((N_NODES, HALF), jnp.float32),
          pltpu.VMEM((_KB,), jnp.int32),
          pltpu.VMEM((_KB,), jnp.int32),
          pltpu.VMEM((_KB + L,), jnp.float32),
          pltpu.VMEM((_KB, HALF), jnp.float32),
          pltpu.SemaphoreType.DMA,
      ],
  )(h1a, h1b, src, dst, w)


_KC = 2000
_EPW_C = N_EDGES // (NC * NS)
_NCH_C = _EPW_C // _KC
_NCH_N = N_NODES // _KC
_ACC_SZ = L * N_GRAPHS


def _scc_body(z0_hbm, z1_hbm, r0_hbm, r1_hbm, batch_hbm, src_hbm, dst_hbm,
              w_hbm, out, srcv, dstv, wv, g0, g1, bdg,
              a0, a1, a2, a3, a4, obuf, sem):
  c = lax.axis_index("c")
  s = lax.axis_index("s")
  wid = c * NS + s
  lane128 = lax.iota(jnp.int32, L) * N_GRAPHS
  ones = jnp.full((L,), 1.0, jnp.float32)

  for a in (a0, a1, a2, a3, a4):
    _zero_vmem(a, _ACC_SZ)

  def echunk(i, _):
    base = wid * _EPW_C + i * _KC
    pltpu.sync_copy(src_hbm.at[pl.ds(base, _KC)], srcv)
    pltpu.sync_copy(dst_hbm.at[pl.ds(base, _KC)], dstv)
    pltpu.sync_copy(w_hbm.at[pl.ds(base, _KC)], wv)
    pltpu.async_copy(z0_hbm.at[srcv], g0, sem).wait()
    pltpu.async_copy(z1_hbm.at[srcv], g1, sem).wait()
    pltpu.async_copy(batch_hbm.at[dstv], bdg, sem).wait()

    def vec(j, _):
      sl = pl.ds(j * L, L)
      idx = bdg[sl] + lane128
      w16 = wv[sl]
      plsc.addupdate_scatter(a0, [idx], g0[sl] * w16)
      plsc.addupdate_scatter(a1, [idx], g1[sl] * w16)
      return 0

    lax.fori_loop(0, _KC // L, vec, 0)
    return 0

  lax.fori_loop(0, _NCH_C, echunk, 0)

  for j in range(-(-_NCH_N // (NC * NS))):
    cid = wid + NC * NS * j

    @pl.when(cid < _NCH_N)
    def _():
      sl_h = pl.ds(cid * _KC, _KC)
      pltpu.sync_copy(batch_hbm.at[sl_h], bdg)
      pltpu.sync_copy(r0_hbm.at[sl_h], g0)
      pltpu.sync_copy(r1_hbm.at[sl_h], g1)

      def vec(j2, _):
        sl = pl.ds(j2 * L, L)
        idx = bdg[sl] + lane128
        plsc.addupdate_scatter(a2, [idx], g0[sl])
        plsc.addupdate_scatter(a3, [idx], g1[sl])
        plsc.addupdate_scatter(a4, [idx], ones)
        return 0

      lax.fori_loop(0, _KC // L, vec, 0)

  for p, a in enumerate((a0, a1, a2, a3, a4)):
    for j in range(N_GRAPHS // L):
      v = jnp.zeros((L,), jnp.float32)
      for lane in range(L):
        v = v + a[pl.ds(lane * N_GRAPHS + j * L, L)]
      obuf[p, pl.ds(j * L, L)] = v

  pltpu.sync_copy(obuf, out.at[wid])


@jax.jit
def _sc_c(z0, z1, r0, r1, batch, src, dst, w):
  return pl.kernel(
      _scc_body,
      out_type=jax.ShapeDtypeStruct((NC * NS, 5, N_GRAPHS), jnp.float32),
      mesh=_MESH,
      compiler_params=pltpu.CompilerParams(needs_layout_passes=False),
      scratch_types=[
          pltpu.VMEM((_KC,), jnp.int32),
          pltpu.VMEM((_KC,), jnp.int32),
          pltpu.VMEM((_KC,), jnp.float32),
          pltpu.VMEM((_KC,), jnp.float32),
          pltpu.VMEM((_KC,), jnp.float32),
          pltpu.VMEM((_KC,), jnp.int32),
          pltpu.VMEM((_ACC_SZ,), jnp.float32),
          pltpu.VMEM((_ACC_SZ,), jnp.float32),
          pltpu.VMEM((_ACC_SZ,), jnp.float32),
          pltpu.VMEM((_ACC_SZ,), jnp.float32),
          pltpu.VMEM((_ACC_SZ,), jnp.float32),
          pltpu.VMEM((5, N_GRAPHS), jnp.float32),
          pltpu.SemaphoreType.DMA,
      ],
  )(z0, z1, r0, r1, batch, src, dst, w)


_BT = 5000
_GT = N_NODES // _BT


def _tc1_body(a0_ref, a1_ref, x_ref, wrel_ref, wroot_ref, b1_ref, ha_ref,
              hb_ref):
  a = a0_ref[:, 0] + a1_ref[:, 0]
  xv = x_ref[:, 0]
  pre = (a[:, None] * wrel_ref[0, :][None, :]
         + xv[:, None] * wroot_ref[0, :][None, :] + b1_ref[0, :][None, :])
  h = jnp.maximum(pre, 0.0)
  ha_ref[...] = h[:, :HALF]
  hb_ref[...] = h[:, HALF:]


@jax.jit
def _tc_1(a0, a1, x, W1_rel, W1_root, b1):
  return pl.pallas_call(
      _tc1_body,
      grid=(_GT,),
      in_specs=[
          pl.BlockSpec((_BT, 1), lambda i: (i, 0)),
          pl.BlockSpec((_BT, 1), lambda i: (i, 0)),
          pl.BlockSpec((_BT, 1), lambda i: (i, 0)),
          pl.BlockSpec((1, HIDDEN), lambda i: (0, 0)),
          pl.BlockSpec((1, HIDDEN), lambda i: (0, 0)),
          pl.BlockSpec((1, HIDDEN), lambda i: (0, 0)),
      ],
      out_specs=[
          pl.BlockSpec((_BT, HALF), lambda i: (i, 0)),
          pl.BlockSpec((_BT, HALF), lambda i: (i, 0)),
      ],
      out_shape=[
          jax.ShapeDtypeStruct((N_NODES, HALF), jnp.float32),
          jax.ShapeDtypeStruct((N_NODES, HALF), jnp.float32),
      ],
  )(a0, a1, x, W1_rel, W1_root, b1)


def _tc2_body(g0_ref, g1_ref, ha_ref, hb_ref, w2rel_ref, w2root_ref, b2_ref,
              w3rel_ref, w3root_ref, linw_ref, z0_ref, z1_ref, r0_ref, r1_ref):
  agg = jnp.concatenate([g0_ref[...], g1_ref[...]], axis=1)
  h1 = jnp.concatenate([ha_ref[...], hb_ref[...]], axis=1)
  pre = (jnp.dot(agg, w2rel_ref[...], preferred_element_type=jnp.float32)
         + jnp.dot(h1, w2root_ref[...], preferred_element_type=jnp.float32)
         + b2_ref[0, :][None, :])
  h2 = jnp.maximum(pre, 0.0)
  wz = jnp.dot(w3rel_ref[...], linw_ref[...],
               preferred_element_type=jnp.float32)
  wr = jnp.dot(w3root_ref[...], linw_ref[...],
               preferred_element_type=jnp.float32)
  z = jnp.dot(h2, wz, preferred_element_type=jnp.float32)
  r = jnp.dot(h2, wr, preferred_element_type=jnp.float32)
  z0_ref[...] = z[:, 0:1]
  z1_ref[...] = z[:, 1:2]
  r0_ref[...] = r[:, 0:1]
  r1_ref[...] = r[:, 1:2]


@jax.jit
def _tc_2(g0, g1, ha, hb, W2_rel, W2_root, b2, W3_rel, W3_root, lin_W):
  full = lambda r, c: pl.BlockSpec((r, c), lambda i: (0, 0))
  return pl.pallas_call(
      _tc2_body,
      grid=(_GT,),
      in_specs=[
          pl.BlockSpec((_BT, HALF), lambda i: (i, 0)),
          pl.BlockSpec((_BT, HALF), lambda i: (i, 0)),
          pl.BlockSpec((_BT, HALF), lambda i: (i, 0)),
          pl.BlockSpec((_BT, HALF), lambda i: (i, 0)),
          full(HIDDEN, HIDDEN),
          full(HIDDEN, HIDDEN),
          full(1, HIDDEN),
          full(HIDDEN, HIDDEN),
          full(HIDDEN, HIDDEN),
          full(HIDDEN, 2),
      ],
      out_specs=[pl.BlockSpec((_BT, 1), lambda i: (i, 0))] * 4,
      out_shape=[jax.ShapeDtypeStruct((N_NODES, 1), jnp.float32)] * 4,
  )(g0, g1, ha, hb, W2_rel, W2_root, b2, W3_rel, W3_root, lin_W)


def _tc3_body(p_ref, b3_ref, linw_ref, linb_ref, out_ref):
  sums = jnp.sum(p_ref[...], axis=0)
  e0, e1, r0, r1, n = (sums[0], sums[1], sums[2], sums[3], sums[4])
  cnt = jnp.maximum(n, 1.0)
  base = jnp.dot(b3_ref[...], linw_ref[...],
                 preferred_element_type=jnp.float32)
  nz = (n > 0.0).astype(jnp.float32)
  col0 = (e0 + r0) / cnt + nz * base[0, 0] + linb_ref[0, 0]
  col1 = (e1 + r1) / cnt + nz * base[0, 1] + linb_ref[0, 1]
  out_ref[...] = jnp.stack([col0, col1], axis=1)


@jax.jit
def _tc_3(partials, b3, lin_W, lin_b):
  return pl.pallas_call(
      _tc3_body,
      in_specs=[
          pl.BlockSpec((NC * NS, 5, N_GRAPHS), lambda: (0, 0, 0)),
          pl.BlockSpec((1, HIDDEN), lambda: (0, 0)),
          pl.BlockSpec((HIDDEN, 2), lambda: (0, 0)),
          pl.BlockSpec((1, 2), lambda: (0, 0)),
      ],
      out_specs=pl.BlockSpec((N_GRAPHS, 2), lambda: (0, 0)),
      out_shape=jax.ShapeDtypeStruct((N_GRAPHS, 2), jnp.float32),
  )(partials, b3, lin_W, lin_b)


def kernel(x, edge_index, batch, edge_weight, W1_rel, b1, W1_root, W2_rel, b2,
           W2_root, W3_rel, b3, W3_root, lin_W, lin_b):
  src = edge_index[0]
  dst = edge_index[1]
  x1d = x[:, 0]

  ap0, ap1 = _sc_a(x1d, src, dst, edge_weight)
  a0 = ap0.reshape(N_NODES, 1)
  a1 = ap1.reshape(N_NODES, 1)
  ha, hb = _tc_1(a0, a1, x, W1_rel, W1_root, b1.reshape(1, HIDDEN))
  g0, g1 = _sc_b(ha, hb, src, dst, edge_weight)
  z0, z1, r0, r1 = _tc_2(g0, g1, ha, hb, W2_rel, W2_root,
                         b2.reshape(1, HIDDEN), W3_rel, W3_root, lin_W)
  partials = _sc_c(z0[:, 0], z1[:, 0], r0[:, 0], r1[:, 0], batch, src, dst,
                   edge_weight)
  return _tc_3(partials, b3.reshape(1, HIDDEN), lin_W, lin_b.reshape(1, 2))


# speedup vs baseline: 20.2105x; 1.4480x over previous
import functools

import jax
import jax.numpy as jnp
from jax import lax
from jax.experimental import pallas as pl
from jax.experimental.pallas import tpu as pltpu
from jax.experimental.pallas import tpu_sc as plsc

N_NODES = 100000
N_EDGES = 1600000
HIDDEN = 32
HALF = 16
N_GRAPHS = 128
NC = 2
NS = 16
L = 16

_MESH = plsc.VectorSubcoreMesh(
    core_axis_name="c", subcore_axis_name="s", num_cores=NC, num_subcores=NS)


def _zero_vmem(ref, n):
  zeros = jnp.zeros((L,), ref.dtype)

  def body(i, _):
    ref[pl.ds(i * L, L)] = zeros
    return 0

  lax.fori_loop(0, n // L, body, 0)


_KA = 2000
_EPW_A = N_EDGES // (NC * NS)
_NCH_A = _EPW_A // _KA
_ZCH_A = N_NODES // _KA


def _sca_body(x_hbm, src_hbm, dst_hbm, w_hbm, out0, out1, acc, srcv, dstv,
              wv, xg, zbuf, sem):
  c = lax.axis_index("c")
  s = lax.axis_index("s")

  _zero_vmem(zbuf, _KA)
  for j in range(-(-_ZCH_A // NS)):
    cid = s + NS * j

    @pl.when(cid < _ZCH_A)
    def _():
      pltpu.sync_copy(zbuf, acc.at[pl.ds(cid * _KA, _KA)])

  plsc.subcore_barrier()

  base0 = (c * NS + s) * _EPW_A

  def chunk(i, _):
    base = base0 + i * _KA
    pltpu.sync_copy(src_hbm.at[pl.ds(base, _KA)], srcv)
    pltpu.sync_copy(dst_hbm.at[pl.ds(base, _KA)], dstv)
    pltpu.sync_copy(w_hbm.at[pl.ds(base, _KA)], wv)
    pltpu.async_copy(x_hbm.at[srcv], xg, sem).wait()

    def scale(j, _):
      sl = pl.ds(j * L, L)
      xg[sl] = xg[sl] * wv[sl]
      return 0

    lax.fori_loop(0, _KA // L, scale, 0)
    pltpu.sync_copy(xg, acc.at[dstv], add=True)
    return 0

  lax.fori_loop(0, _NCH_A, chunk, 0)
  plsc.subcore_barrier()

  for j in range(-(-_ZCH_A // NS)):
    cid = s + NS * j

    @pl.when(cid < _ZCH_A)
    def _():
      sl = pl.ds(cid * _KA, _KA)
      pltpu.sync_copy(acc.at[sl], xg)

      @pl.when(c == 0)
      def _():
        pltpu.sync_copy(xg, out0.at[sl])

      @pl.when(c == 1)
      def _():
        pltpu.sync_copy(xg, out1.at[sl])


@jax.jit
def _sc_a(x1d, src, dst, w):
  return pl.kernel(
      _sca_body,
      out_type=(
          jax.ShapeDtypeStruct((N_NODES,), jnp.float32),
          jax.ShapeDtypeStruct((N_NODES,), jnp.float32),
      ),
      mesh=_MESH,
      scratch_types=[
          pltpu.VMEM_SHARED((N_NODES,), jnp.float32),
          pltpu.VMEM((_KA,), jnp.int32),
          pltpu.VMEM((_KA,), jnp.int32),
          pltpu.VMEM((_KA,), jnp.float32),
          pltpu.VMEM((_KA,), jnp.float32),
          pltpu.VMEM((_KA,), jnp.float32),
          pltpu.SemaphoreType.DMA,
      ],
  )(x1d, src, dst, w)


_KB = 400
_EPT_B = N_EDGES // NS
_NCH_B = _EPT_B // _KB
_ZCH_B = N_NODES // _KB


def _scb_body(h1a_hbm, h1b_hbm, src_hbm, dst_hbm, w_hbm, out0, out1,
              acc, srcvA, dstvA, wvA, rowsA, srcvB, dstvB, wvB, rowsB,
              semA, semB, semSA, semSB, semL):
  c = lax.axis_index("c")
  s = lax.axis_index("s")

  def zrow(i, _):
    rowsA[i, :] = jnp.zeros((L,), jnp.float32)
    return 0

  lax.fori_loop(0, _KB, zrow, 0)

  for j in range(-(-_ZCH_B // NS)):
    cid = s + NS * j

    @pl.when(cid < _ZCH_B)
    def _():
      pltpu.sync_copy(rowsA, acc.at[pl.ds(cid * _KB, _KB)])

  plsc.subcore_barrier()

  def loads(i, srcv, dstv, wv):
    base = s * _EPT_B + i * _KB
    d1 = pltpu.async_copy(src_hbm.at[pl.ds(base, _KB)], srcv, semL)
    d2 = pltpu.async_copy(dst_hbm.at[pl.ds(base, _KB)], dstv, semL)
    d3 = pltpu.async_copy(w_hbm.at[pl.ds(base, _KB)], wv, semL)
    d1.wait(); d2.wait(); d3.wait()

  def g_start(srcv, rows, sem):
    @pl.when(c == 0)
    def _():
      pltpu.async_copy(h1a_hbm.at[srcv], rows, sem)

    @pl.when(c == 1)
    def _():
      pltpu.async_copy(h1b_hbm.at[srcv], rows, sem)

  def g_wait(srcv, rows, sem):
    @pl.when(c == 0)
    def _():
      pltpu.make_async_copy(h1a_hbm.at[srcv], rows, sem).wait()

    @pl.when(c == 1)
    def _():
      pltpu.make_async_copy(h1b_hbm.at[srcv], rows, sem).wait()

  def sc_start(rows, dstv, sem):
    pltpu.async_copy(rows, acc.at[dstv], sem, add=True)

  def sc_wait(rows, dstv, sem):
    pltpu.make_async_copy(rows, acc.at[dstv], sem).wait()

  def scale(rows, wv):
    def grp(g, _):
      base = g * L
      w16 = wv[pl.ds(base, L)]
      for j in range(L):
        rows[base + j, :] = rows[base + j, :] * w16[j]
      return 0

    lax.fori_loop(0, _KB // L, grp, 0)

  loads(0, srcvA, dstvA, wvA)
  g_start(srcvA, rowsA, semA)
  loads(1, srcvB, dstvB, wvB)

  def pair(i2, _):
    i = 2 * i2
    more = i2 < _NCH_B // 2 - 1
    g_wait(srcvA, rowsA, semA)
    g_start(srcvB, rowsB, semB)
    scale(rowsA, wvA)
    sc_start(rowsA, dstvA, semSA)
    g_wait(srcvB, rowsB, semB)
    sc_wait(rowsA, dstvA, semSA)

    @pl.when(more)
    def _():
      loads(i + 2, srcvA, dstvA, wvA)
      g_start(srcvA, rowsA, semA)

    scale(rowsB, wvB)
    sc_start(rowsB, dstvB, semSB)
    sc_wait(rowsB, dstvB, semSB)

    @pl.when(more)
    def _():
      loads(i + 3, srcvB, dstvB, wvB)

    return 0

  lax.fori_loop(0, _NCH_B // 2, pair, 0)
  plsc.subcore_barrier()

  for j in range(-(-_ZCH_B // NS)):
    cid = s + NS * j

    @pl.when(cid < _ZCH_B)
    def _():
      sl = pl.ds(cid * _KB, _KB)
      pltpu.sync_copy(acc.at[sl], rowsA)

      @pl.when(c == 0)
      def _():
        pltpu.sync_copy(rowsA, out0.at[sl])

      @pl.when(c == 1)
      def _():
        pltpu.sync_copy(rowsA, out1.at[sl])


@jax.jit
def _sc_b(h1a, h1b, src, dst, w):
  return pl.kernel(
      _scb_body,
      out_type=(
          jax.ShapeDtypeStruct((N_NODES, HALF), jnp.float32),
          jax.ShapeDtypeStruct((N_NODES, HALF), jnp.float32),
      ),
      mesh=_MESH,
      compiler_params=pltpu.CompilerParams(use_tc_tiling_on_sc=False),
      scratch_types=[
          pltpu.VMEM_SHARED((N_NODES, HALF), jnp.float32),
          pltpu.VMEM((_KB,), jnp.int32),
          pltpu.VMEM((_KB,), jnp.int32),
          pltpu.VMEM((_KB,), jnp.float32),
          pltpu.VMEM((_KB, HALF), jnp.float32),
          pltpu.VMEM((_KB,), jnp.int32),
          pltpu.VMEM((_KB,), jnp.int32),
          pltpu.VMEM((_KB,), jnp.float32),
          pltpu.VMEM((_KB, HALF), jnp.float32),
          pltpu.SemaphoreType.DMA,
          pltpu.SemaphoreType.DMA,
          pltpu.SemaphoreType.DMA,
          pltpu.SemaphoreType.DMA,
          pltpu.SemaphoreType.DMA,
      ],
  )(h1a, h1b, src, dst, w)


_KC = 2000
_EPW_C = N_EDGES // (NC * NS)
_NCH_C = _EPW_C // _KC
_NCH_N = N_NODES // _KC
_ACC_SZ = L * N_GRAPHS


def _scc_body(z0_hbm, z1_hbm, r0_hbm, r1_hbm, batch_hbm, src_hbm, dst_hbm,
              w_hbm, out, srcv, dstv, wv, g0, g1, bdg,
              a0, a1, a2, a3, a4, obuf, sem):
  c = lax.axis_index("c")
  s = lax.axis_index("s")
  wid = c * NS + s
  lane128 = lax.iota(jnp.int32, L) * N_GRAPHS
  ones = jnp.full((L,), 1.0, jnp.float32)

  for a in (a0, a1, a2, a3, a4):
    _zero_vmem(a, _ACC_SZ)

  def echunk(i, _):
    base = wid * _EPW_C + i * _KC
    pltpu.sync_copy(src_hbm.at[pl.ds(base, _KC)], srcv)
    pltpu.sync_copy(dst_hbm.at[pl.ds(base, _KC)], dstv)
    pltpu.sync_copy(w_hbm.at[pl.ds(base, _KC)], wv)
    pltpu.async_copy(z0_hbm.at[srcv], g0, sem).wait()
    pltpu.async_copy(z1_hbm.at[srcv], g1, sem).wait()
    pltpu.async_copy(batch_hbm.at[dstv], bdg, sem).wait()

    def vec(j, _):
      sl = pl.ds(j * L, L)
      idx = bdg[sl] + lane128
      w16 = wv[sl]
      plsc.addupdate_scatter(a0, [idx], g0[sl] * w16)
      plsc.addupdate_scatter(a1, [idx], g1[sl] * w16)
      return 0

    lax.fori_loop(0, _KC // L, vec, 0)
    return 0

  lax.fori_loop(0, _NCH_C, echunk, 0)

  for j in range(-(-_NCH_N // (NC * NS))):
    cid = wid + NC * NS * j

    @pl.when(cid < _NCH_N)
    def _():
      sl_h = pl.ds(cid * _KC, _KC)
      pltpu.sync_copy(batch_hbm.at[sl_h], bdg)
      pltpu.sync_copy(r0_hbm.at[sl_h], g0)
      pltpu.sync_copy(r1_hbm.at[sl_h], g1)

      def vec(j2, _):
        sl = pl.ds(j2 * L, L)
        idx = bdg[sl] + lane128
        plsc.addupdate_scatter(a2, [idx], g0[sl])
        plsc.addupdate_scatter(a3, [idx], g1[sl])
        plsc.addupdate_scatter(a4, [idx], ones)
        return 0

      lax.fori_loop(0, _KC // L, vec, 0)

  for p, a in enumerate((a0, a1, a2, a3, a4)):
    for j in range(N_GRAPHS // L):
      v = jnp.zeros((L,), jnp.float32)
      for lane in range(L):
        v = v + a[pl.ds(lane * N_GRAPHS + j * L, L)]
      obuf[p, pl.ds(j * L, L)] = v

  pltpu.sync_copy(obuf, out.at[wid])


@jax.jit
def _sc_c(z0, z1, r0, r1, batch, src, dst, w):
  return pl.kernel(
      _scc_body,
      out_type=jax.ShapeDtypeStruct((NC * NS, 5, N_GRAPHS), jnp.float32),
      mesh=_MESH,
      compiler_params=pltpu.CompilerParams(needs_layout_passes=False),
      scratch_types=[
          pltpu.VMEM((_KC,), jnp.int32),
          pltpu.VMEM((_KC,), jnp.int32),
          pltpu.VMEM((_KC,), jnp.float32),
          pltpu.VMEM((_KC,), jnp.float32),
          pltpu.VMEM((_KC,), jnp.float32),
          pltpu.VMEM((_KC,), jnp.int32),
          pltpu.VMEM((_ACC_SZ,), jnp.float32),
          pltpu.VMEM((_ACC_SZ,), jnp.float32),
          pltpu.VMEM((_ACC_SZ,), jnp.float32),
          pltpu.VMEM((_ACC_SZ,), jnp.float32),
          pltpu.VMEM((_ACC_SZ,), jnp.float32),
          pltpu.VMEM((5, N_GRAPHS), jnp.float32),
          pltpu.SemaphoreType.DMA,
      ],
  )(z0, z1, r0, r1, batch, src, dst, w)


_BT = 5000
_GT = N_NODES // _BT


def _tc1_body(a0_ref, a1_ref, x_ref, wrel_ref, wroot_ref, b1_ref, ha_ref,
              hb_ref):
  a = a0_ref[:, 0] + a1_ref[:, 0]
  xv = x_ref[:, 0]
  pre = (a[:, None] * wrel_ref[0, :][None, :]
         + xv[:, None] * wroot_ref[0, :][None, :] + b1_ref[0, :][None, :])
  h = jnp.maximum(pre, 0.0)
  ha_ref[...] = h[:, :HALF]
  hb_ref[...] = h[:, HALF:]


@jax.jit
def _tc_1(a0, a1, x, W1_rel, W1_root, b1):
  return pl.pallas_call(
      _tc1_body,
      grid=(_GT,),
      in_specs=[
          pl.BlockSpec((_BT, 1), lambda i: (i, 0)),
          pl.BlockSpec((_BT, 1), lambda i: (i, 0)),
          pl.BlockSpec((_BT, 1), lambda i: (i, 0)),
          pl.BlockSpec((1, HIDDEN), lambda i: (0, 0)),
          pl.BlockSpec((1, HIDDEN), lambda i: (0, 0)),
          pl.BlockSpec((1, HIDDEN), lambda i: (0, 0)),
      ],
      out_specs=[
          pl.BlockSpec((_BT, HALF), lambda i: (i, 0)),
          pl.BlockSpec((_BT, HALF), lambda i: (i, 0)),
      ],
      out_shape=[
          jax.ShapeDtypeStruct((N_NODES, HALF), jnp.float32),
          jax.ShapeDtypeStruct((N_NODES, HALF), jnp.float32),
      ],
  )(a0, a1, x, W1_rel, W1_root, b1)


def _tc2_body(g0_ref, g1_ref, ha_ref, hb_ref, w2rel_ref, w2root_ref, b2_ref,
              w3rel_ref, w3root_ref, linw_ref, z0_ref, z1_ref, r0_ref, r1_ref):
  agg = jnp.concatenate([g0_ref[...], g1_ref[...]], axis=1)
  h1 = jnp.concatenate([ha_ref[...], hb_ref[...]], axis=1)
  pre = (jnp.dot(agg, w2rel_ref[...], preferred_element_type=jnp.float32)
         + jnp.dot(h1, w2root_ref[...], preferred_element_type=jnp.float32)
         + b2_ref[0, :][None, :])
  h2 = jnp.maximum(pre, 0.0)
  wz = jnp.dot(w3rel_ref[...], linw_ref[...],
               preferred_element_type=jnp.float32)
  wr = jnp.dot(w3root_ref[...], linw_ref[...],
               preferred_element_type=jnp.float32)
  z = jnp.dot(h2, wz, preferred_element_type=jnp.float32)
  r = jnp.dot(h2, wr, preferred_element_type=jnp.float32)
  z0_ref[...] = z[:, 0:1]
  z1_ref[...] = z[:, 1:2]
  r0_ref[...] = r[:, 0:1]
  r1_ref[...] = r[:, 1:2]


@jax.jit
def _tc_2(g0, g1, ha, hb, W2_rel, W2_root, b2, W3_rel, W3_root, lin_W):
  full = lambda r, c: pl.BlockSpec((r, c), lambda i: (0, 0))
  return pl.pallas_call(
      _tc2_body,
      grid=(_GT,),
      in_specs=[
          pl.BlockSpec((_BT, HALF), lambda i: (i, 0)),
          pl.BlockSpec((_BT, HALF), lambda i: (i, 0)),
          pl.BlockSpec((_BT, HALF), lambda i: (i, 0)),
          pl.BlockSpec((_BT, HALF), lambda i: (i, 0)),
          full(HIDDEN, HIDDEN),
          full(HIDDEN, HIDDEN),
          full(1, HIDDEN),
          full(HIDDEN, HIDDEN),
          full(HIDDEN, HIDDEN),
          full(HIDDEN, 2),
      ],
      out_specs=[pl.BlockSpec((_BT, 1), lambda i: (i, 0))] * 4,
      out_shape=[jax.ShapeDtypeStruct((N_NODES, 1), jnp.float32)] * 4,
  )(g0, g1, ha, hb, W2_rel, W2_root, b2, W3_rel, W3_root, lin_W)


def _tc3_body(p_ref, b3_ref, linw_ref, linb_ref, out_ref):
  sums = jnp.sum(p_ref[...], axis=0)
  e0, e1, r0, r1, n = (sums[0], sums[1], sums[2], sums[3], sums[4])
  cnt = jnp.maximum(n, 1.0)
  base = jnp.dot(b3_ref[...], linw_ref[...],
                 preferred_element_type=jnp.float32)
  nz = (n > 0.0).astype(jnp.float32)
  col0 = (e0 + r0) / cnt + nz * base[0, 0] + linb_ref[0, 0]
  col1 = (e1 + r1) / cnt + nz * base[0, 1] + linb_ref[0, 1]
  out_ref[...] = jnp.stack([col0, col1], axis=1)


@jax.jit
def _tc_3(partials, b3, lin_W, lin_b):
  return pl.pallas_call(
      _tc3_body,
      in_specs=[
          pl.BlockSpec((NC * NS, 5, N_GRAPHS), lambda: (0, 0, 0)),
          pl.BlockSpec((1, HIDDEN), lambda: (0, 0)),
          pl.BlockSpec((HIDDEN, 2), lambda: (0, 0)),
          pl.BlockSpec((1, 2), lambda: (0, 0)),
      ],
      out_specs=pl.BlockSpec((N_GRAPHS, 2), lambda: (0, 0)),
      out_shape=jax.ShapeDtypeStruct((N_GRAPHS, 2), jnp.float32),
  )(partials, b3, lin_W, lin_b)


def kernel(x, edge_index, batch, edge_weight, W1_rel, b1, W1_root, W2_rel, b2,
           W2_root, W3_rel, b3, W3_root, lin_W, lin_b):
  src = edge_index[0]
  dst = edge_index[1]
  x1d = x[:, 0]

  ap0, ap1 = _sc_a(x1d, src, dst, edge_weight)
  a0 = ap0.reshape(N_NODES, 1)
  a1 = ap1.reshape(N_NODES, 1)
  ha, hb = _tc_1(a0, a1, x, W1_rel, W1_root, b1.reshape(1, HIDDEN))
  g0, g1 = _sc_b(ha, hb, src, dst, edge_weight)
  z0, z1, r0, r1 = _tc_2(g0, g1, ha, hb, W2_rel, W2_root,
                         b2.reshape(1, HIDDEN), W3_rel, W3_root, lin_W)
  partials = _sc_c(z0[:, 0], z1[:, 0], r0[:, 0], r1[:, 0], batch, src, dst,
                   edge_weight)
  return _tc_3(partials, b3.reshape(1, HIDDEN), lin_W, lin_b.reshape(1, 2))


# speedup vs baseline: 23.0557x; 1.1408x over previous
import functools

import jax
import jax.numpy as jnp
from jax import lax
from jax.experimental import pallas as pl
from jax.experimental.pallas import tpu as pltpu
from jax.experimental.pallas import tpu_sc as plsc

N_NODES = 100000
N_EDGES = 1600000
HIDDEN = 32
HALF = 16
N_GRAPHS = 128
NC = 2
NS = 16
L = 16

_MESH = plsc.VectorSubcoreMesh(
    core_axis_name="c", subcore_axis_name="s", num_cores=NC, num_subcores=NS)


def _zero_vmem(ref, n):
  zeros = jnp.zeros((L,), ref.dtype)

  def body(i, _):
    ref[pl.ds(i * L, L)] = zeros
    return 0

  lax.fori_loop(0, n // L, body, 0)


_KA = 2000
_EPW_A = N_EDGES // (NC * NS)
_NCH_A = _EPW_A // _KA
_ZCH_A = N_NODES // _KA


def _sca_body(x_hbm, src_hbm, dst_hbm, w_hbm, out0, out1, acc,
              srcvA, dstvA, wvA, xgA, srcvB, dstvB, wvB, xgB,
              semA, semB, semSA, semSB, semL):
  c = lax.axis_index("c")
  s = lax.axis_index("s")

  _zero_vmem(xgA, _KA)
  for j in range(-(-_ZCH_A // NS)):
    cid = s + NS * j

    @pl.when(cid < _ZCH_A)
    def _():
      pltpu.sync_copy(xgA, acc.at[pl.ds(cid * _KA, _KA)])

  plsc.subcore_barrier()

  base0 = (c * NS + s) * _EPW_A

  def loads(i, srcv, dstv, wv):
    base = base0 + i * _KA
    d1 = pltpu.async_copy(src_hbm.at[pl.ds(base, _KA)], srcv, semL)
    d2 = pltpu.async_copy(dst_hbm.at[pl.ds(base, _KA)], dstv, semL)
    d3 = pltpu.async_copy(w_hbm.at[pl.ds(base, _KA)], wv, semL)
    d1.wait(); d2.wait(); d3.wait()

  def scale(xg, wv):
    def body(j, _):
      sl = pl.ds(j * L, L)
      xg[sl] = xg[sl] * wv[sl]
      return 0

    lax.fori_loop(0, _KA // L, body, 0)

  loads(0, srcvA, dstvA, wvA)
  pltpu.async_copy(x_hbm.at[srcvA], xgA, semA).wait()
  scale(xgA, wvA)
  pltpu.sync_copy(xgA, acc.at[dstvA], add=True)

  loads(1, srcvA, dstvA, wvA)
  pltpu.async_copy(x_hbm.at[srcvA], xgA, semA)
  loads(2, srcvB, dstvB, wvB)

  def pair(i2, _):
    i = 1 + 2 * i2
    more = i2 < (_NCH_A - 1) // 2 - 1
    pltpu.make_async_copy(x_hbm.at[srcvA], xgA, semA).wait()
    pltpu.async_copy(x_hbm.at[srcvB], xgB, semB)
    scale(xgA, wvA)
    pltpu.async_copy(xgA, acc.at[dstvA], semSA, add=True)
    pltpu.make_async_copy(x_hbm.at[srcvB], xgB, semB).wait()
    pltpu.make_async_copy(xgA, acc.at[dstvA], semSA).wait()

    @pl.when(more)
    def _():
      loads(i + 2, srcvA, dstvA, wvA)
      pltpu.async_copy(x_hbm.at[srcvA], xgA, semA)

    scale(xgB, wvB)
    pltpu.async_copy(xgB, acc.at[dstvB], semSB, add=True)
    pltpu.make_async_copy(xgB, acc.at[dstvB], semSB).wait()

    @pl.when(more)
    def _():
      loads(i + 3, srcvB, dstvB, wvB)

    return 0

  lax.fori_loop(0, (_NCH_A - 1) // 2, pair, 0)
  plsc.subcore_barrier()

  for j in range(-(-_ZCH_A // NS)):
    cid = s + NS * j

    @pl.when(cid < _ZCH_A)
    def _():
      sl = pl.ds(cid * _KA, _KA)
      pltpu.sync_copy(acc.at[sl], xgA)

      @pl.when(c == 0)
      def _():
        pltpu.sync_copy(xgA, out0.at[sl])

      @pl.when(c == 1)
      def _():
        pltpu.sync_copy(xgA, out1.at[sl])


@jax.jit
def _sc_a(x1d, src, dst, w):
  return pl.kernel(
      _sca_body,
      out_type=(
          jax.ShapeDtypeStruct((N_NODES,), jnp.float32),
          jax.ShapeDtypeStruct((N_NODES,), jnp.float32),
      ),
      mesh=_MESH,
      scratch_types=[
          pltpu.VMEM_SHARED((N_NODES,), jnp.float32),
          pltpu.VMEM((_KA,), jnp.int32),
          pltpu.VMEM((_KA,), jnp.int32),
          pltpu.VMEM((_KA,), jnp.float32),
          pltpu.VMEM((_KA,), jnp.float32),
          pltpu.VMEM((_KA,), jnp.int32),
          pltpu.VMEM((_KA,), jnp.int32),
          pltpu.VMEM((_KA,), jnp.float32),
          pltpu.VMEM((_KA,), jnp.float32),
          pltpu.SemaphoreType.DMA,
          pltpu.SemaphoreType.DMA,
          pltpu.SemaphoreType.DMA,
          pltpu.SemaphoreType.DMA,
          pltpu.SemaphoreType.DMA,
      ],
  )(x1d, src, dst, w)


_KB = 400
_EPT_B = N_EDGES // NS
_NCH_B = _EPT_B // _KB
_ZCH_B = N_NODES // _KB


def _scb_body(h1a_hbm, h1b_hbm, src_hbm, dst_hbm, w_hbm, out0, out1,
              acc, srcvA, dstvA, wvA, rowsA, srcvB, dstvB, wvB, rowsB,
              semA, semB, semSA, semSB, semL):
  c = lax.axis_index("c")
  s = lax.axis_index("s")

  def zrow(i, _):
    rowsA[i, :] = jnp.zeros((L,), jnp.float32)
    return 0

  lax.fori_loop(0, _KB, zrow, 0)

  for j in range(-(-_ZCH_B // NS)):
    cid = s + NS * j

    @pl.when(cid < _ZCH_B)
    def _():
      pltpu.sync_copy(rowsA, acc.at[pl.ds(cid * _KB, _KB)])

  plsc.subcore_barrier()

  def loads(i, srcv, dstv, wv):
    base = s * _EPT_B + i * _KB
    d1 = pltpu.async_copy(src_hbm.at[pl.ds(base, _KB)], srcv, semL)
    d2 = pltpu.async_copy(dst_hbm.at[pl.ds(base, _KB)], dstv, semL)
    d3 = pltpu.async_copy(w_hbm.at[pl.ds(base, _KB)], wv, semL)
    d1.wait(); d2.wait(); d3.wait()

  def g_start(srcv, rows, sem):
    @pl.when(c == 0)
    def _():
      pltpu.async_copy(h1a_hbm.at[srcv], rows, sem)

    @pl.when(c == 1)
    def _():
      pltpu.async_copy(h1b_hbm.at[srcv], rows, sem)

  def g_wait(srcv, rows, sem):
    @pl.when(c == 0)
    def _():
      pltpu.make_async_copy(h1a_hbm.at[srcv], rows, sem).wait()

    @pl.when(c == 1)
    def _():
      pltpu.make_async_copy(h1b_hbm.at[srcv], rows, sem).wait()

  def sc_start(rows, dstv, sem):
    pltpu.async_copy(rows, acc.at[dstv], sem, add=True)

  def sc_wait(rows, dstv, sem):
    pltpu.make_async_copy(rows, acc.at[dstv], sem).wait()

  def scale(rows, wv):
    def grp(g, _):
      base = g * L
      w16 = wv[pl.ds(base, L)]
      for j in range(L):
        rows[base + j, :] = rows[base + j, :] * w16[j]
      return 0

    lax.fori_loop(0, _KB // L, grp, 0)

  loads(0, srcvA, dstvA, wvA)
  g_start(srcvA, rowsA, semA)
  loads(1, srcvB, dstvB, wvB)

  def pair(i2, _):
    i = 2 * i2
    more = i2 < _NCH_B // 2 - 1
    g_wait(srcvA, rowsA, semA)
    g_start(srcvB, rowsB, semB)
    scale(rowsA, wvA)
    sc_start(rowsA, dstvA, semSA)
    g_wait(srcvB, rowsB, semB)
    sc_wait(rowsA, dstvA, semSA)

    @pl.when(more)
    def _():
      loads(i + 2, srcvA, dstvA, wvA)
      g_start(srcvA, rowsA, semA)

    scale(rowsB, wvB)
    sc_start(rowsB, dstvB, semSB)
    sc_wait(rowsB, dstvB, semSB)

    @pl.when(more)
    def _():
      loads(i + 3, srcvB, dstvB, wvB)

    return 0

  lax.fori_loop(0, _NCH_B // 2, pair, 0)
  plsc.subcore_barrier()

  for j in range(-(-_ZCH_B // NS)):
    cid = s + NS * j

    @pl.when(cid < _ZCH_B)
    def _():
      sl = pl.ds(cid * _KB, _KB)
      pltpu.sync_copy(acc.at[sl], rowsA)

      @pl.when(c == 0)
      def _():
        pltpu.sync_copy(rowsA, out0.at[sl])

      @pl.when(c == 1)
      def _():
        pltpu.sync_copy(rowsA, out1.at[sl])


@jax.jit
def _sc_b(h1a, h1b, src, dst, w):
  return pl.kernel(
      _scb_body,
      out_type=(
          jax.ShapeDtypeStruct((N_NODES, HALF), jnp.float32),
          jax.ShapeDtypeStruct((N_NODES, HALF), jnp.float32),
      ),
      mesh=_MESH,
      compiler_params=pltpu.CompilerParams(use_tc_tiling_on_sc=False),
      scratch_types=[
          pltpu.VMEM_SHARED((N_NODES, HALF), jnp.float32),
          pltpu.VMEM((_KB,), jnp.int32),
          pltpu.VMEM((_KB,), jnp.int32),
          pltpu.VMEM((_KB,), jnp.float32),
          pltpu.VMEM((_KB, HALF), jnp.float32),
          pltpu.VMEM((_KB,), jnp.int32),
          pltpu.VMEM((_KB,), jnp.int32),
          pltpu.VMEM((_KB,), jnp.float32),
          pltpu.VMEM((_KB, HALF), jnp.float32),
          pltpu.SemaphoreType.DMA,
          pltpu.SemaphoreType.DMA,
          pltpu.SemaphoreType.DMA,
          pltpu.SemaphoreType.DMA,
          pltpu.SemaphoreType.DMA,
      ],
  )(h1a, h1b, src, dst, w)


_KC = 2000
_EPW_C = N_EDGES // (NC * NS)
_NCH_C = _EPW_C // _KC
_NCH_N = N_NODES // _KC
_ACC_SZ = L * N_GRAPHS


def _scc_body(z0_hbm, z1_hbm, r0_hbm, r1_hbm, batch_hbm, src_hbm, dst_hbm,
              w_hbm, out,
              srcvA, dstvA, wvA, g0A, g1A, bdgA,
              srcvB, dstvB, wvB, g0B, g1B, bdgB,
              a0, a1, a2, a3, a4, obuf, semGA, semGB, semL):
  c = lax.axis_index("c")
  s = lax.axis_index("s")
  wid = c * NS + s
  lane128 = lax.iota(jnp.int32, L) * N_GRAPHS
  ones = jnp.full((L,), 1.0, jnp.float32)

  for a in (a0, a1, a2, a3, a4):
    _zero_vmem(a, _ACC_SZ)

  def eloads(i, srcv, dstv, wv):
    base = wid * _EPW_C + i * _KC
    d1 = pltpu.async_copy(src_hbm.at[pl.ds(base, _KC)], srcv, semL)
    d2 = pltpu.async_copy(dst_hbm.at[pl.ds(base, _KC)], dstv, semL)
    d3 = pltpu.async_copy(w_hbm.at[pl.ds(base, _KC)], wv, semL)
    d1.wait(); d2.wait(); d3.wait()

  def g_start3(srcv, dstv, g0, g1, bdg, sem):
    pltpu.async_copy(z0_hbm.at[srcv], g0, sem)
    pltpu.async_copy(z1_hbm.at[srcv], g1, sem)
    pltpu.async_copy(batch_hbm.at[dstv], bdg, sem)

  def g_wait3(srcv, dstv, g0, g1, bdg, sem):
    pltpu.make_async_copy(z0_hbm.at[srcv], g0, sem).wait()
    pltpu.make_async_copy(z1_hbm.at[srcv], g1, sem).wait()
    pltpu.make_async_copy(batch_hbm.at[dstv], bdg, sem).wait()

  def accum(g0, g1, bdg, wv):
    def vec(j, _):
      sl = pl.ds(j * L, L)
      idx = bdg[sl] + lane128
      w16 = wv[sl]
      plsc.addupdate_scatter(a0, [idx], g0[sl] * w16)
      plsc.addupdate_scatter(a1, [idx], g1[sl] * w16)
      return 0

    lax.fori_loop(0, _KC // L, vec, 0)

  eloads(0, srcvA, dstvA, wvA)
  g_start3(srcvA, dstvA, g0A, g1A, bdgA, semGA)
  g_wait3(srcvA, dstvA, g0A, g1A, bdgA, semGA)
  accum(g0A, g1A, bdgA, wvA)

  eloads(1, srcvA, dstvA, wvA)
  g_start3(srcvA, dstvA, g0A, g1A, bdgA, semGA)
  eloads(2, srcvB, dstvB, wvB)

  def pair(i2, _):
    i = 1 + 2 * i2
    more = i2 < (_NCH_C - 1) // 2 - 1
    g_wait3(srcvA, dstvA, g0A, g1A, bdgA, semGA)
    g_start3(srcvB, dstvB, g0B, g1B, bdgB, semGB)
    accum(g0A, g1A, bdgA, wvA)

    @pl.when(more)
    def _():
      eloads(i + 2, srcvA, dstvA, wvA)
      g_start3(srcvA, dstvA, g0A, g1A, bdgA, semGA)

    g_wait3(srcvB, dstvB, g0B, g1B, bdgB, semGB)
    accum(g0B, g1B, bdgB, wvB)

    @pl.when(more)
    def _():
      eloads(i + 3, srcvB, dstvB, wvB)

    return 0

  lax.fori_loop(0, (_NCH_C - 1) // 2, pair, 0)

  for j in range(-(-_NCH_N // (NC * NS))):
    cid = wid + NC * NS * j

    @pl.when(cid < _NCH_N)
    def _():
      sl_h = pl.ds(cid * _KC, _KC)
      d1 = pltpu.async_copy(batch_hbm.at[sl_h], bdgA, semL)
      d2 = pltpu.async_copy(r0_hbm.at[sl_h], g0A, semL)
      d3 = pltpu.async_copy(r1_hbm.at[sl_h], g1A, semL)
      d1.wait(); d2.wait(); d3.wait()

      def vec(j2, _):
        sl = pl.ds(j2 * L, L)
        idx = bdgA[sl] + lane128
        plsc.addupdate_scatter(a2, [idx], g0A[sl])
        plsc.addupdate_scatter(a3, [idx], g1A[sl])
        plsc.addupdate_scatter(a4, [idx], ones)
        return 0

      lax.fori_loop(0, _KC // L, vec, 0)

  for p, a in enumerate((a0, a1, a2, a3, a4)):
    for j in range(N_GRAPHS // L):
      v = jnp.zeros((L,), jnp.float32)
      for lane in range(L):
        v = v + a[pl.ds(lane * N_GRAPHS + j * L, L)]
      obuf[p, pl.ds(j * L, L)] = v

  pltpu.sync_copy(obuf, out.at[wid])


@jax.jit
def _sc_c(z0, z1, r0, r1, batch, src, dst, w):
  return pl.kernel(
      _scc_body,
      out_type=jax.ShapeDtypeStruct((NC * NS, 5, N_GRAPHS), jnp.float32),
      mesh=_MESH,
      compiler_params=pltpu.CompilerParams(needs_layout_passes=False),
      scratch_types=[
          pltpu.VMEM((_KC,), jnp.int32),
          pltpu.VMEM((_KC,), jnp.int32),
          pltpu.VMEM((_KC,), jnp.float32),
          pltpu.VMEM((_KC,), jnp.float32),
          pltpu.VMEM((_KC,), jnp.float32),
          pltpu.VMEM((_KC,), jnp.int32),
          pltpu.VMEM((_KC,), jnp.int32),
          pltpu.VMEM((_KC,), jnp.int32),
          pltpu.VMEM((_KC,), jnp.float32),
          pltpu.VMEM((_KC,), jnp.float32),
          pltpu.VMEM((_KC,), jnp.float32),
          pltpu.VMEM((_KC,), jnp.int32),
          pltpu.VMEM((_ACC_SZ,), jnp.float32),
          pltpu.VMEM((_ACC_SZ,), jnp.float32),
          pltpu.VMEM((_ACC_SZ,), jnp.float32),
          pltpu.VMEM((_ACC_SZ,), jnp.float32),
          pltpu.VMEM((_ACC_SZ,), jnp.float32),
          pltpu.VMEM((5, N_GRAPHS), jnp.float32),
          pltpu.SemaphoreType.DMA,
          pltpu.SemaphoreType.DMA,
          pltpu.SemaphoreType.DMA,
      ],
  )(z0, z1, r0, r1, batch, src, dst, w)


_BT = 5000
_GT = N_NODES // _BT


def _tc1_body(a0_ref, a1_ref, x_ref, wrel_ref, wroot_ref, b1_ref, ha_ref,
              hb_ref):
  a = a0_ref[:, 0] + a1_ref[:, 0]
  xv = x_ref[:, 0]
  pre = (a[:, None] * wrel_ref[0, :][None, :]
         + xv[:, None] * wroot_ref[0, :][None, :] + b1_ref[0, :][None, :])
  h = jnp.maximum(pre, 0.0)
  ha_ref[...] = h[:, :HALF]
  hb_ref[...] = h[:, HALF:]


@jax.jit
def _tc_1(a0, a1, x, W1_rel, W1_root, b1):
  return pl.pallas_call(
      _tc1_body,
      grid=(_GT,),
      in_specs=[
          pl.BlockSpec((_BT, 1), lambda i: (i, 0)),
          pl.BlockSpec((_BT, 1), lambda i: (i, 0)),
          pl.BlockSpec((_BT, 1), lambda i: (i, 0)),
          pl.BlockSpec((1, HIDDEN), lambda i: (0, 0)),
          pl.BlockSpec((1, HIDDEN), lambda i: (0, 0)),
          pl.BlockSpec((1, HIDDEN), lambda i: (0, 0)),
      ],
      out_specs=[
          pl.BlockSpec((_BT, HALF), lambda i: (i, 0)),
          pl.BlockSpec((_BT, HALF), lambda i: (i, 0)),
      ],
      out_shape=[
          jax.ShapeDtypeStruct((N_NODES, HALF), jnp.float32),
          jax.ShapeDtypeStruct((N_NODES, HALF), jnp.float32),
      ],
  )(a0, a1, x, W1_rel, W1_root, b1)


def _tc2_body(g0_ref, g1_ref, ha_ref, hb_ref, w2rel_ref, w2root_ref, b2_ref,
              w3rel_ref, w3root_ref, linw_ref, z0_ref, z1_ref, r0_ref, r1_ref):
  agg = jnp.concatenate([g0_ref[...], g1_ref[...]], axis=1)
  h1 = jnp.concatenate([ha_ref[...], hb_ref[...]], axis=1)
  pre = (jnp.dot(agg, w2rel_ref[...], preferred_element_type=jnp.float32)
         + jnp.dot(h1, w2root_ref[...], preferred_element_type=jnp.float32)
         + b2_ref[0, :][None, :])
  h2 = jnp.maximum(pre, 0.0)
  wz = jnp.dot(w3rel_ref[...], linw_ref[...],
               preferred_element_type=jnp.float32)
  wr = jnp.dot(w3root_ref[...], linw_ref[...],
               preferred_element_type=jnp.float32)
  z = jnp.dot(h2, wz, preferred_element_type=jnp.float32)
  r = jnp.dot(h2, wr, preferred_element_type=jnp.float32)
  z0_ref[...] = z[:, 0:1]
  z1_ref[...] = z[:, 1:2]
  r0_ref[...] = r[:, 0:1]
  r1_ref[...] = r[:, 1:2]


@jax.jit
def _tc_2(g0, g1, ha, hb, W2_rel, W2_root, b2, W3_rel, W3_root, lin_W):
  full = lambda r, c: pl.BlockSpec((r, c), lambda i: (0, 0))
  return pl.pallas_call(
      _tc2_body,
      grid=(_GT,),
      in_specs=[
          pl.BlockSpec((_BT, HALF), lambda i: (i, 0)),
          pl.BlockSpec((_BT, HALF), lambda i: (i, 0)),
          pl.BlockSpec((_BT, HALF), lambda i: (i, 0)),
          pl.BlockSpec((_BT, HALF), lambda i: (i, 0)),
          full(HIDDEN, HIDDEN),
          full(HIDDEN, HIDDEN),
          full(1, HIDDEN),
          full(HIDDEN, HIDDEN),
          full(HIDDEN, HIDDEN),
          full(HIDDEN, 2),
      ],
      out_specs=[pl.BlockSpec((_BT, 1), lambda i: (i, 0))] * 4,
      out_shape=[jax.ShapeDtypeStruct((N_NODES, 1), jnp.float32)] * 4,
  )(g0, g1, ha, hb, W2_rel, W2_root, b2, W3_rel, W3_root, lin_W)


def _tc3_body(p_ref, b3_ref, linw_ref, linb_ref, out_ref):
  sums = jnp.sum(p_ref[...], axis=0)
  e0, e1, r0, r1, n = (sums[0], sums[1], sums[2], sums[3], sums[4])
  cnt = jnp.maximum(n, 1.0)
  base = jnp.dot(b3_ref[...], linw_ref[...],
                 preferred_element_type=jnp.float32)
  nz = (n > 0.0).astype(jnp.float32)
  col0 = (e0 + r0) / cnt + nz * base[0, 0] + linb_ref[0, 0]
  col1 = (e1 + r1) / cnt + nz * base[0, 1] + linb_ref[0, 1]
  out_ref[...] = jnp.stack([col0, col1], axis=1)


@jax.jit
def _tc_3(partials, b3, lin_W, lin_b):
  return pl.pallas_call(
      _tc3_body,
      in_specs=[
          pl.BlockSpec((NC * NS, 5, N_GRAPHS), lambda: (0, 0, 0)),
          pl.BlockSpec((1, HIDDEN), lambda: (0, 0)),
          pl.BlockSpec((HIDDEN, 2), lambda: (0, 0)),
          pl.BlockSpec((1, 2), lambda: (0, 0)),
      ],
      out_specs=pl.BlockSpec((N_GRAPHS, 2), lambda: (0, 0)),
      out_shape=jax.ShapeDtypeStruct((N_GRAPHS, 2), jnp.float32),
  )(partials, b3, lin_W, lin_b)


def kernel(x, edge_index, batch, edge_weight, W1_rel, b1, W1_root, W2_rel, b2,
           W2_root, W3_rel, b3, W3_root, lin_W, lin_b):
  src = edge_index[0]
  dst = edge_index[1]
  x1d = x[:, 0]

  ap0, ap1 = _sc_a(x1d, src, dst, edge_weight)
  a0 = ap0.reshape(N_NODES, 1)
  a1 = ap1.reshape(N_NODES, 1)
  ha, hb = _tc_1(a0, a1, x, W1_rel, W1_root, b1.reshape(1, HIDDEN))
  g0, g1 = _sc_b(ha, hb, src, dst, edge_weight)
  z0, z1, r0, r1 = _tc_2(g0, g1, ha, hb, W2_rel, W2_root,
                         b2.reshape(1, HIDDEN), W3_rel, W3_root, lin_W)
  partials = _sc_c(z0[:, 0], z1[:, 0], r0[:, 0], r1[:, 0], batch, src, dst,
                   edge_weight)
  return _tc_3(partials, b3.reshape(1, HIDDEN), lin_W, lin_b.reshape(1, 2))


# speedup vs baseline: 27.5065x; 1.1930x over previous
import functools

import jax
import jax.numpy as jnp
from jax import lax
from jax.experimental import pallas as pl
from jax.experimental.pallas import tpu as pltpu
from jax.experimental.pallas import tpu_sc as plsc

N_NODES = 100000
N_EDGES = 1600000
HIDDEN = 32
HALF = 16
N_GRAPHS = 128
NC = 2
NS = 16
L = 16

_MESH = plsc.VectorSubcoreMesh(
    core_axis_name="c", subcore_axis_name="s", num_cores=NC, num_subcores=NS)


def _zero_vmem(ref, n):
  zeros = jnp.zeros((L,), ref.dtype)

  def body(i, _):
    ref[pl.ds(i * L, L)] = zeros
    return 0

  lax.fori_loop(0, n // L, body, 0)


_KA = 2000
_EPW_A = N_EDGES // (NC * NS)
_NCH_A = _EPW_A // _KA
_ZCH_A = N_NODES // _KA


def _sca_body(x_hbm, ei_hbm, w_hbm, out0, out1, acc,
              srcvA, dstvA, wvA, xgA, srcvB, dstvB, wvB, xgB,
              semA, semB, semSA, semSB, semL):
  c = lax.axis_index("c")
  s = lax.axis_index("s")

  _zero_vmem(xgA, _KA)
  for j in range(-(-_ZCH_A // NS)):
    cid = s + NS * j

    @pl.when(cid < _ZCH_A)
    def _():
      pltpu.sync_copy(xgA, acc.at[pl.ds(cid * _KA, _KA)])

  plsc.subcore_barrier()

  base0 = (c * NS + s) * _EPW_A

  def loads(i, srcv, dstv, wv):
    base = base0 + i * _KA
    d1 = pltpu.async_copy(ei_hbm.at[0, pl.ds(base, _KA)], srcv, semL)
    d2 = pltpu.async_copy(ei_hbm.at[1, pl.ds(base, _KA)], dstv, semL)
    d3 = pltpu.async_copy(w_hbm.at[pl.ds(base, _KA)], wv, semL)
    d1.wait(); d2.wait(); d3.wait()

  def scale(xg, wv):
    def body(j, _):
      sl = pl.ds(j * L, L)
      xg[sl] = xg[sl] * wv[sl]
      return 0

    lax.fori_loop(0, _KA // L, body, 0)

  loads(0, srcvA, dstvA, wvA)
  pltpu.async_copy(x_hbm.at[srcvA], xgA, semA).wait()
  scale(xgA, wvA)
  pltpu.sync_copy(xgA, acc.at[dstvA], add=True)

  loads(1, srcvA, dstvA, wvA)
  pltpu.async_copy(x_hbm.at[srcvA], xgA, semA)
  loads(2, srcvB, dstvB, wvB)

  def pair(i2, _):
    i = 1 + 2 * i2
    more = i2 < (_NCH_A - 1) // 2 - 1
    pltpu.make_async_copy(x_hbm.at[srcvA], xgA, semA).wait()
    pltpu.async_copy(x_hbm.at[srcvB], xgB, semB)
    scale(xgA, wvA)
    pltpu.async_copy(xgA, acc.at[dstvA], semSA, add=True)
    pltpu.make_async_copy(x_hbm.at[srcvB], xgB, semB).wait()
    pltpu.make_async_copy(xgA, acc.at[dstvA], semSA).wait()

    @pl.when(more)
    def _():
      loads(i + 2, srcvA, dstvA, wvA)
      pltpu.async_copy(x_hbm.at[srcvA], xgA, semA)

    scale(xgB, wvB)
    pltpu.async_copy(xgB, acc.at[dstvB], semSB, add=True)
    pltpu.make_async_copy(xgB, acc.at[dstvB], semSB).wait()

    @pl.when(more)
    def _():
      loads(i + 3, srcvB, dstvB, wvB)

    return 0

  lax.fori_loop(0, (_NCH_A - 1) // 2, pair, 0)
  plsc.subcore_barrier()

  for j in range(-(-_ZCH_A // NS)):
    cid = s + NS * j

    @pl.when(cid < _ZCH_A)
    def _():
      sl = pl.ds(cid * _KA, _KA)
      pltpu.sync_copy(acc.at[sl], xgA)

      @pl.when(c == 0)
      def _():
        pltpu.sync_copy(xgA, out0.at[sl])

      @pl.when(c == 1)
      def _():
        pltpu.sync_copy(xgA, out1.at[sl])


@jax.jit
def _sc_a(x1d, ei, w):
  return pl.kernel(
      _sca_body,
      out_type=(
          jax.ShapeDtypeStruct((N_NODES,), jnp.float32),
          jax.ShapeDtypeStruct((N_NODES,), jnp.float32),
      ),
      mesh=_MESH,
      compiler_params=pltpu.CompilerParams(use_tc_tiling_on_sc=False),
      scratch_types=[
          pltpu.VMEM_SHARED((N_NODES,), jnp.float32),
          pltpu.VMEM((_KA,), jnp.int32),
          pltpu.VMEM((_KA,), jnp.int32),
          pltpu.VMEM((_KA,), jnp.float32),
          pltpu.VMEM((_KA,), jnp.float32),
          pltpu.VMEM((_KA,), jnp.int32),
          pltpu.VMEM((_KA,), jnp.int32),
          pltpu.VMEM((_KA,), jnp.float32),
          pltpu.VMEM((_KA,), jnp.float32),
          pltpu.SemaphoreType.DMA,
          pltpu.SemaphoreType.DMA,
          pltpu.SemaphoreType.DMA,
          pltpu.SemaphoreType.DMA,
          pltpu.SemaphoreType.DMA,
      ],
  )(x1d, ei, w)


_KB = 400
_EPT_B = N_EDGES // NS
_NCH_B = _EPT_B // _KB
_ZCH_B = N_NODES // _KB


def _scb_body(h1a_hbm, h1b_hbm, ei_hbm, w_hbm, out0, out1,
              acc, srcvA, dstvA, wvA, rowsA, srcvB, dstvB, wvB, rowsB,
              semA, semB, semSA, semSB, semL):
  c = lax.axis_index("c")
  s = lax.axis_index("s")

  def zrow(i, _):
    rowsA[i, :] = jnp.zeros((L,), jnp.float32)
    return 0

  lax.fori_loop(0, _KB, zrow, 0)

  for j in range(-(-_ZCH_B // NS)):
    cid = s + NS * j

    @pl.when(cid < _ZCH_B)
    def _():
      pltpu.sync_copy(rowsA, acc.at[pl.ds(cid * _KB, _KB)])

  plsc.subcore_barrier()

  def loads(i, srcv, dstv, wv):
    base = s * _EPT_B + i * _KB
    d1 = pltpu.async_copy(ei_hbm.at[0, pl.ds(base, _KB)], srcv, semL)
    d2 = pltpu.async_copy(ei_hbm.at[1, pl.ds(base, _KB)], dstv, semL)
    d3 = pltpu.async_copy(w_hbm.at[pl.ds(base, _KB)], wv, semL)
    d1.wait(); d2.wait(); d3.wait()

  def g_start(srcv, rows, sem):
    @pl.when(c == 0)
    def _():
      pltpu.async_copy(h1a_hbm.at[srcv], rows, sem)

    @pl.when(c == 1)
    def _():
      pltpu.async_copy(h1b_hbm.at[srcv], rows, sem)

  def g_wait(srcv, rows, sem):
    @pl.when(c == 0)
    def _():
      pltpu.make_async_copy(h1a_hbm.at[srcv], rows, sem).wait()

    @pl.when(c == 1)
    def _():
      pltpu.make_async_copy(h1b_hbm.at[srcv], rows, sem).wait()

  def sc_start(rows, dstv, sem):
    pltpu.async_copy(rows, acc.at[dstv], sem, add=True)

  def sc_wait(rows, dstv, sem):
    pltpu.make_async_copy(rows, acc.at[dstv], sem).wait()

  def scale(rows, wv):
    def grp(g, _):
      base = g * L
      w16 = wv[pl.ds(base, L)]
      for j in range(L):
        rows[base + j, :] = rows[base + j, :] * w16[j]
      return 0

    lax.fori_loop(0, _KB // L, grp, 0)

  loads(0, srcvA, dstvA, wvA)
  g_start(srcvA, rowsA, semA)
  loads(1, srcvB, dstvB, wvB)

  def pair(i2, _):
    i = 2 * i2
    more = i2 < _NCH_B // 2 - 1
    g_wait(srcvA, rowsA, semA)
    g_start(srcvB, rowsB, semB)
    scale(rowsA, wvA)
    sc_start(rowsA, dstvA, semSA)
    g_wait(srcvB, rowsB, semB)
    sc_wait(rowsA, dstvA, semSA)

    @pl.when(more)
    def _():
      loads(i + 2, srcvA, dstvA, wvA)
      g_start(srcvA, rowsA, semA)

    scale(rowsB, wvB)
    sc_start(rowsB, dstvB, semSB)
    sc_wait(rowsB, dstvB, semSB)

    @pl.when(more)
    def _():
      loads(i + 3, srcvB, dstvB, wvB)

    return 0

  lax.fori_loop(0, _NCH_B // 2, pair, 0)
  plsc.subcore_barrier()

  for j in range(-(-_ZCH_B // NS)):
    cid = s + NS * j

    @pl.when(cid < _ZCH_B)
    def _():
      sl = pl.ds(cid * _KB, _KB)
      pltpu.sync_copy(acc.at[sl], rowsA)

      @pl.when(c == 0)
      def _():
        pltpu.sync_copy(rowsA, out0.at[sl])

      @pl.when(c == 1)
      def _():
        pltpu.sync_copy(rowsA, out1.at[sl])


@jax.jit
def _sc_b(h1a, h1b, ei, w):
  return pl.kernel(
      _scb_body,
      out_type=(
          jax.ShapeDtypeStruct((N_NODES, HALF), jnp.float32),
          jax.ShapeDtypeStruct((N_NODES, HALF), jnp.float32),
      ),
      mesh=_MESH,
      compiler_params=pltpu.CompilerParams(use_tc_tiling_on_sc=False),
      scratch_types=[
          pltpu.VMEM_SHARED((N_NODES, HALF), jnp.float32),
          pltpu.VMEM((_KB,), jnp.int32),
          pltpu.VMEM((_KB,), jnp.int32),
          pltpu.VMEM((_KB,), jnp.float32),
          pltpu.VMEM((_KB, HALF), jnp.float32),
          pltpu.VMEM((_KB,), jnp.int32),
          pltpu.VMEM((_KB,), jnp.int32),
          pltpu.VMEM((_KB,), jnp.float32),
          pltpu.VMEM((_KB, HALF), jnp.float32),
          pltpu.SemaphoreType.DMA,
          pltpu.SemaphoreType.DMA,
          pltpu.SemaphoreType.DMA,
          pltpu.SemaphoreType.DMA,
          pltpu.SemaphoreType.DMA,
      ],
  )(h1a, h1b, ei, w)


_KC = 2000
_EPW_C = N_EDGES // (NC * NS)
_NCH_C = _EPW_C // _KC
_NCH_N = N_NODES // _KC
_ACC_SZ = L * N_GRAPHS


def _scc_body(z0_hbm, z1_hbm, r0_hbm, r1_hbm, batch_hbm, ei_hbm,
              w_hbm, out,
              srcvA, dstvA, wvA, g0A, g1A, bdgA,
              srcvB, dstvB, wvB, g0B, g1B, bdgB,
              a0, a1, a2, a3, a4, obuf, semGA, semGB, semL):
  c = lax.axis_index("c")
  s = lax.axis_index("s")
  wid = c * NS + s
  lane128 = lax.iota(jnp.int32, L) * N_GRAPHS
  ones = jnp.full((L,), 1.0, jnp.float32)

  for a in (a0, a1, a2, a3, a4):
    _zero_vmem(a, _ACC_SZ)

  def eloads(i, srcv, dstv, wv):
    base = wid * _EPW_C + i * _KC
    d1 = pltpu.async_copy(ei_hbm.at[0, pl.ds(base, _KC)], srcv, semL)
    d2 = pltpu.async_copy(ei_hbm.at[1, pl.ds(base, _KC)], dstv, semL)
    d3 = pltpu.async_copy(w_hbm.at[pl.ds(base, _KC)], wv, semL)
    d1.wait(); d2.wait(); d3.wait()

  def g_start3(srcv, dstv, g0, g1, bdg, sem):
    pltpu.async_copy(z0_hbm.at[srcv], g0, sem)
    pltpu.async_copy(z1_hbm.at[srcv], g1, sem)
    pltpu.async_copy(batch_hbm.at[dstv], bdg, sem)

  def g_wait3(srcv, dstv, g0, g1, bdg, sem):
    pltpu.make_async_copy(z0_hbm.at[srcv], g0, sem).wait()
    pltpu.make_async_copy(z1_hbm.at[srcv], g1, sem).wait()
    pltpu.make_async_copy(batch_hbm.at[dstv], bdg, sem).wait()

  def accum(g0, g1, bdg, wv):
    def vec(j, _):
      sl = pl.ds(j * L, L)
      idx = bdg[sl] + lane128
      w16 = wv[sl]
      plsc.addupdate_scatter(a0, [idx], g0[sl] * w16)
      plsc.addupdate_scatter(a1, [idx], g1[sl] * w16)
      return 0

    lax.fori_loop(0, _KC // L, vec, 0)

  eloads(0, srcvA, dstvA, wvA)
  g_start3(srcvA, dstvA, g0A, g1A, bdgA, semGA)
  g_wait3(srcvA, dstvA, g0A, g1A, bdgA, semGA)
  accum(g0A, g1A, bdgA, wvA)

  eloads(1, srcvA, dstvA, wvA)
  g_start3(srcvA, dstvA, g0A, g1A, bdgA, semGA)
  eloads(2, srcvB, dstvB, wvB)

  def pair(i2, _):
    i = 1 + 2 * i2
    more = i2 < (_NCH_C - 1) // 2 - 1
    g_wait3(srcvA, dstvA, g0A, g1A, bdgA, semGA)
    g_start3(srcvB, dstvB, g0B, g1B, bdgB, semGB)
    accum(g0A, g1A, bdgA, wvA)

    @pl.when(more)
    def _():
      eloads(i + 2, srcvA, dstvA, wvA)
      g_start3(srcvA, dstvA, g0A, g1A, bdgA, semGA)

    g_wait3(srcvB, dstvB, g0B, g1B, bdgB, semGB)
    accum(g0B, g1B, bdgB, wvB)

    @pl.when(more)
    def _():
      eloads(i + 3, srcvB, dstvB, wvB)

    return 0

  lax.fori_loop(0, (_NCH_C - 1) // 2, pair, 0)

  for j in range(-(-_NCH_N // (NC * NS))):
    cid = wid + NC * NS * j

    @pl.when(cid < _NCH_N)
    def _():
      sl_h = pl.ds(cid * _KC, _KC)
      d1 = pltpu.async_copy(batch_hbm.at[sl_h], bdgA, semL)
      d2 = pltpu.async_copy(r0_hbm.at[sl_h], g0A, semL)
      d3 = pltpu.async_copy(r1_hbm.at[sl_h], g1A, semL)
      d1.wait(); d2.wait(); d3.wait()

      def vec(j2, _):
        sl = pl.ds(j2 * L, L)
        idx = bdgA[sl] + lane128
        plsc.addupdate_scatter(a2, [idx], g0A[sl])
        plsc.addupdate_scatter(a3, [idx], g1A[sl])
        plsc.addupdate_scatter(a4, [idx], ones)
        return 0

      lax.fori_loop(0, _KC // L, vec, 0)

  for p, a in enumerate((a0, a1, a2, a3, a4)):
    for j in range(N_GRAPHS // L):
      v = jnp.zeros((L,), jnp.float32)
      for lane in range(L):
        v = v + a[pl.ds(lane * N_GRAPHS + j * L, L)]
      obuf[p, pl.ds(j * L, L)] = v

  pltpu.sync_copy(obuf, out.at[wid])


@jax.jit
def _sc_c(z0, z1, r0, r1, batch, ei, w):
  return pl.kernel(
      _scc_body,
      out_type=jax.ShapeDtypeStruct((NC * NS, 5, N_GRAPHS), jnp.float32),
      mesh=_MESH,
      compiler_params=pltpu.CompilerParams(
          needs_layout_passes=False, use_tc_tiling_on_sc=False),
      scratch_types=[
          pltpu.VMEM((_KC,), jnp.int32),
          pltpu.VMEM((_KC,), jnp.int32),
          pltpu.VMEM((_KC,), jnp.float32),
          pltpu.VMEM((_KC,), jnp.float32),
          pltpu.VMEM((_KC,), jnp.float32),
          pltpu.VMEM((_KC,), jnp.int32),
          pltpu.VMEM((_KC,), jnp.int32),
          pltpu.VMEM((_KC,), jnp.int32),
          pltpu.VMEM((_KC,), jnp.float32),
          pltpu.VMEM((_KC,), jnp.float32),
          pltpu.VMEM((_KC,), jnp.float32),
          pltpu.VMEM((_KC,), jnp.int32),
          pltpu.VMEM((_ACC_SZ,), jnp.float32),
          pltpu.VMEM((_ACC_SZ,), jnp.float32),
          pltpu.VMEM((_ACC_SZ,), jnp.float32),
          pltpu.VMEM((_ACC_SZ,), jnp.float32),
          pltpu.VMEM((_ACC_SZ,), jnp.float32),
          pltpu.VMEM((5, N_GRAPHS), jnp.float32),
          pltpu.SemaphoreType.DMA,
          pltpu.SemaphoreType.DMA,
          pltpu.SemaphoreType.DMA,
      ],
  )(z0, z1, r0, r1, batch, ei, w)


_BT = 6144
_GT = -(-N_NODES // _BT)
_NPAD = _BT * _GT


def _tc1_body(a0_ref, a1_ref, x_ref, wrel_ref, wroot_ref, b1_ref, ha_ref,
              hb_ref):
  a = a0_ref[...] + a1_ref[...]
  xv = x_ref[...]
  pre = (a[:, None] * wrel_ref[0, :][None, :]
         + xv[:, None] * wroot_ref[0, :][None, :] + b1_ref[0, :][None, :])
  h = jnp.maximum(pre, 0.0)
  ha_ref[...] = h[:, :HALF]
  hb_ref[...] = h[:, HALF:]


@jax.jit
def _tc_1(a0, a1, x1d, W1_rel, W1_root, b1):
  return pl.pallas_call(
      _tc1_body,
      grid=(_GT,),
      in_specs=[
          pl.BlockSpec((_BT,), lambda i: (i,)),
          pl.BlockSpec((_BT,), lambda i: (i,)),
          pl.BlockSpec((_BT,), lambda i: (i,)),
          pl.BlockSpec((1, HIDDEN), lambda i: (0, 0)),
          pl.BlockSpec((1, HIDDEN), lambda i: (0, 0)),
          pl.BlockSpec((1, HIDDEN), lambda i: (0, 0)),
      ],
      out_specs=[
          pl.BlockSpec((_BT, HALF), lambda i: (i, 0)),
          pl.BlockSpec((_BT, HALF), lambda i: (i, 0)),
      ],
      out_shape=[
          jax.ShapeDtypeStruct((N_NODES, HALF), jnp.float32),
          jax.ShapeDtypeStruct((N_NODES, HALF), jnp.float32),
      ],
  )(a0, a1, x1d, W1_rel, W1_root, b1)


def _tc2_body(g0_ref, g1_ref, ha_ref, hb_ref, w2rel_ref, w2root_ref, b2_ref,
              w3rel_ref, w3root_ref, linw_ref, z0_ref, z1_ref, r0_ref, r1_ref):
  dot = lambda a, b: jnp.dot(a, b, preferred_element_type=jnp.float32)
  pre = (dot(g0_ref[...], w2rel_ref[:HALF, :])
         + dot(g1_ref[...], w2rel_ref[HALF:, :])
         + dot(ha_ref[...], w2root_ref[:HALF, :])
         + dot(hb_ref[...], w2root_ref[HALF:, :])
         + b2_ref[0, :][None, :])
  h2 = jnp.maximum(pre, 0.0)
  wz = dot(w3rel_ref[...], linw_ref[...])
  wr = dot(w3root_ref[...], linw_ref[...])
  z = dot(h2, wz)
  r = dot(h2, wr)
  z0_ref[...] = z[:, 0]
  z1_ref[...] = z[:, 1]
  r0_ref[...] = r[:, 0]
  r1_ref[...] = r[:, 1]


@jax.jit
def _tc_2(g0, g1, ha, hb, W2_rel, W2_root, b2, W3_rel, W3_root, lin_W):
  full = lambda r, c: pl.BlockSpec((r, c), lambda i: (0, 0))
  return pl.pallas_call(
      _tc2_body,
      grid=(_GT,),
      in_specs=[
          pl.BlockSpec((_BT, HALF), lambda i: (i, 0)),
          pl.BlockSpec((_BT, HALF), lambda i: (i, 0)),
          pl.BlockSpec((_BT, HALF), lambda i: (i, 0)),
          pl.BlockSpec((_BT, HALF), lambda i: (i, 0)),
          full(HIDDEN, HIDDEN),
          full(HIDDEN, HIDDEN),
          full(1, HIDDEN),
          full(HIDDEN, HIDDEN),
          full(HIDDEN, HIDDEN),
          full(HIDDEN, 2),
      ],
      out_specs=[pl.BlockSpec((_BT,), lambda i: (i,))] * 4,
      out_shape=[jax.ShapeDtypeStruct((_NPAD,), jnp.float32)] * 4,
  )(g0, g1, ha, hb, W2_rel, W2_root, b2, W3_rel, W3_root, lin_W)


def _tc3_body(p_ref, b3_ref, linw_ref, linb_ref, out_ref):
  sums = jnp.sum(p_ref[...], axis=0)
  e0, e1, r0, r1, n = (sums[0], sums[1], sums[2], sums[3], sums[4])
  cnt = jnp.maximum(n, 1.0)
  base = jnp.dot(b3_ref[...], linw_ref[...],
                 preferred_element_type=jnp.float32)
  nz = (n > 0.0).astype(jnp.float32)
  col0 = (e0 + r0) / cnt + nz * base[0, 0] + linb_ref[0, 0]
  col1 = (e1 + r1) / cnt + nz * base[0, 1] + linb_ref[0, 1]
  out_ref[...] = jnp.stack([col0, col1], axis=1)


@jax.jit
def _tc_3(partials, b3, lin_W, lin_b):
  return pl.pallas_call(
      _tc3_body,
      in_specs=[
          pl.BlockSpec((NC * NS, 5, N_GRAPHS), lambda: (0, 0, 0)),
          pl.BlockSpec((1, HIDDEN), lambda: (0, 0)),
          pl.BlockSpec((HIDDEN, 2), lambda: (0, 0)),
          pl.BlockSpec((1, 2), lambda: (0, 0)),
      ],
      out_specs=pl.BlockSpec((N_GRAPHS, 2), lambda: (0, 0)),
      out_shape=jax.ShapeDtypeStruct((N_GRAPHS, 2), jnp.float32),
  )(partials, b3, lin_W, lin_b)


def kernel(x, edge_index, batch, edge_weight, W1_rel, b1, W1_root, W2_rel, b2,
           W2_root, W3_rel, b3, W3_root, lin_W, lin_b):
  x1d = x[:, 0]

  ap0, ap1 = _sc_a(x1d, edge_index, edge_weight)
  ha, hb = _tc_1(ap0, ap1, x1d, W1_rel, W1_root, b1.reshape(1, HIDDEN))
  g0, g1 = _sc_b(ha, hb, edge_index, edge_weight)
  z0, z1, r0, r1 = _tc_2(g0, g1, ha, hb, W2_rel, W2_root,
                         b2.reshape(1, HIDDEN), W3_rel, W3_root, lin_W)
  partials = _sc_c(z0, z1, r0, r1, batch, edge_index, edge_weight)
  return _tc_3(partials, b3.reshape(1, HIDDEN), lin_W, lin_b.reshape(1, 2))


# speedup vs baseline: 30.4410x; 1.1067x over previous
import functools

import jax
import jax.numpy as jnp
from jax import lax
from jax.experimental import pallas as pl
from jax.experimental.pallas import tpu as pltpu
from jax.experimental.pallas import tpu_sc as plsc

N_NODES = 100000
N_EDGES = 1600000
HIDDEN = 32
HALF = 16
N_GRAPHS = 128
NC = 2
NS = 16
L = 16

_MESH = plsc.VectorSubcoreMesh(
    core_axis_name="c", subcore_axis_name="s", num_cores=NC, num_subcores=NS)


def _zero_vmem(ref, n):
  zeros = jnp.zeros((L,), ref.dtype)

  def body(i, _):
    ref[pl.ds(i * L, L)] = zeros
    return 0

  lax.fori_loop(0, n // L, body, 0)


_KA = 2000
_EPW_A = N_EDGES // (NC * NS)
_NCH_A = _EPW_A // _KA
_ZCH_A = N_NODES // _KA


def _sca_body(x_hbm, ei_hbm, w_hbm, out0, out1, acc,
              srcvA, dstvA, wvA, xgA, srcvB, dstvB, wvB, xgB,
              semA, semB, semSA, semSB, semL):
  c = lax.axis_index("c")
  s = lax.axis_index("s")

  _zero_vmem(xgA, _KA)
  for j in range(-(-_ZCH_A // NS)):
    cid = s + NS * j

    @pl.when(cid < _ZCH_A)
    def _():
      pltpu.sync_copy(xgA, acc.at[pl.ds(cid * _KA, _KA)])

  plsc.subcore_barrier()

  base0 = (c * NS + s) * _EPW_A

  def loads(i, srcv, dstv, wv):
    base = base0 + i * _KA
    d1 = pltpu.async_copy(ei_hbm.at[0, pl.ds(base, _KA)], srcv, semL)
    d2 = pltpu.async_copy(ei_hbm.at[1, pl.ds(base, _KA)], dstv, semL)
    d3 = pltpu.async_copy(w_hbm.at[pl.ds(base, _KA)], wv, semL)
    d1.wait(); d2.wait(); d3.wait()

  def scale(xg, wv):
    def body(j, _):
      sl = pl.ds(j * L, L)
      xg[sl] = xg[sl] * wv[sl]
      return 0

    lax.fori_loop(0, _KA // L, body, 0)

  loads(0, srcvA, dstvA, wvA)
  pltpu.async_copy(x_hbm.at[srcvA], xgA, semA).wait()
  scale(xgA, wvA)
  pltpu.sync_copy(xgA, acc.at[dstvA], add=True)

  loads(1, srcvA, dstvA, wvA)
  pltpu.async_copy(x_hbm.at[srcvA], xgA, semA)
  loads(2, srcvB, dstvB, wvB)

  def pair(i2, _):
    i = 1 + 2 * i2
    more = i2 < (_NCH_A - 1) // 2 - 1
    pltpu.make_async_copy(x_hbm.at[srcvA], xgA, semA).wait()
    pltpu.async_copy(x_hbm.at[srcvB], xgB, semB)
    scale(xgA, wvA)
    pltpu.async_copy(xgA, acc.at[dstvA], semSA, add=True)
    pltpu.make_async_copy(x_hbm.at[srcvB], xgB, semB).wait()
    pltpu.make_async_copy(xgA, acc.at[dstvA], semSA).wait()

    @pl.when(more)
    def _():
      loads(i + 2, srcvA, dstvA, wvA)
      pltpu.async_copy(x_hbm.at[srcvA], xgA, semA)

    scale(xgB, wvB)
    pltpu.async_copy(xgB, acc.at[dstvB], semSB, add=True)
    pltpu.make_async_copy(xgB, acc.at[dstvB], semSB).wait()

    @pl.when(more)
    def _():
      loads(i + 3, srcvB, dstvB, wvB)

    return 0

  lax.fori_loop(0, (_NCH_A - 1) // 2, pair, 0)
  plsc.subcore_barrier()

  for j in range(-(-_ZCH_A // NS)):
    cid = s + NS * j

    @pl.when(cid < _ZCH_A)
    def _():
      sl = pl.ds(cid * _KA, _KA)
      pltpu.sync_copy(acc.at[sl], xgA)

      @pl.when(c == 0)
      def _():
        pltpu.sync_copy(xgA, out0.at[sl])

      @pl.when(c == 1)
      def _():
        pltpu.sync_copy(xgA, out1.at[sl])


@jax.jit
def _sc_a(x1d, ei, w):
  return pl.kernel(
      _sca_body,
      out_type=(
          jax.ShapeDtypeStruct((N_NODES,), jnp.float32),
          jax.ShapeDtypeStruct((N_NODES,), jnp.float32),
      ),
      mesh=_MESH,
      compiler_params=pltpu.CompilerParams(use_tc_tiling_on_sc=False),
      scratch_types=[
          pltpu.VMEM_SHARED((N_NODES,), jnp.float32),
          pltpu.VMEM((_KA,), jnp.int32),
          pltpu.VMEM((_KA,), jnp.int32),
          pltpu.VMEM((_KA,), jnp.float32),
          pltpu.VMEM((_KA,), jnp.float32),
          pltpu.VMEM((_KA,), jnp.int32),
          pltpu.VMEM((_KA,), jnp.int32),
          pltpu.VMEM((_KA,), jnp.float32),
          pltpu.VMEM((_KA,), jnp.float32),
          pltpu.SemaphoreType.DMA,
          pltpu.SemaphoreType.DMA,
          pltpu.SemaphoreType.DMA,
          pltpu.SemaphoreType.DMA,
          pltpu.SemaphoreType.DMA,
      ],
  )(x1d, ei, w)


_KB = 800
_EPT_B = N_EDGES // NS
_NCH_B = _EPT_B // _KB
_ZCH_B = N_NODES // _KB


def _scb_body(h1a_hbm, h1b_hbm, ei_hbm, w_hbm, out0, out1,
              acc, srcvA, dstvA, wvA, rowsA, srcvB, dstvB, wvB, rowsB,
              semA, semB, semSA, semSB, semL):
  c = lax.axis_index("c")
  s = lax.axis_index("s")

  def zrow(i, _):
    rowsA[i, :] = jnp.zeros((L,), jnp.float32)
    return 0

  lax.fori_loop(0, _KB, zrow, 0)

  for j in range(-(-_ZCH_B // NS)):
    cid = s + NS * j

    @pl.when(cid < _ZCH_B)
    def _():
      pltpu.sync_copy(rowsA, acc.at[pl.ds(cid * _KB, _KB)])

  plsc.subcore_barrier()

  def loads(i, srcv, dstv, wv):
    base = s * _EPT_B + i * _KB
    d1 = pltpu.async_copy(ei_hbm.at[0, pl.ds(base, _KB)], srcv, semL)
    d2 = pltpu.async_copy(ei_hbm.at[1, pl.ds(base, _KB)], dstv, semL)
    d3 = pltpu.async_copy(w_hbm.at[pl.ds(base, _KB)], wv, semL)
    d1.wait(); d2.wait(); d3.wait()

  def g_start(srcv, rows, sem):
    @pl.when(c == 0)
    def _():
      pltpu.async_copy(h1a_hbm.at[srcv], rows, sem)

    @pl.when(c == 1)
    def _():
      pltpu.async_copy(h1b_hbm.at[srcv], rows, sem)

  def g_wait(srcv, rows, sem):
    @pl.when(c == 0)
    def _():
      pltpu.make_async_copy(h1a_hbm.at[srcv], rows, sem).wait()

    @pl.when(c == 1)
    def _():
      pltpu.make_async_copy(h1b_hbm.at[srcv], rows, sem).wait()

  def sc_start(rows, dstv, sem):
    pltpu.async_copy(rows, acc.at[dstv], sem, add=True)

  def sc_wait(rows, dstv, sem):
    pltpu.make_async_copy(rows, acc.at[dstv], sem).wait()

  def scale(rows, wv):
    def grp(g, _):
      base = g * L
      w16 = wv[pl.ds(base, L)]
      for j in range(L):
        rows[base + j, :] = rows[base + j, :] * w16[j]
      return 0

    lax.fori_loop(0, _KB // L, grp, 0)

  loads(0, srcvA, dstvA, wvA)
  g_start(srcvA, rowsA, semA)
  g_wait(srcvA, rowsA, semA)
  scale(rowsA, wvA)
  sc_start(rowsA, dstvA, semSA)
  sc_wait(rowsA, dstvA, semSA)

  loads(1, srcvA, dstvA, wvA)
  g_start(srcvA, rowsA, semA)
  loads(2, srcvB, dstvB, wvB)

  def pair(i2, _):
    i = 1 + 2 * i2
    more = i2 < (_NCH_B - 1) // 2 - 1
    g_wait(srcvA, rowsA, semA)
    g_start(srcvB, rowsB, semB)
    scale(rowsA, wvA)
    sc_start(rowsA, dstvA, semSA)
    g_wait(srcvB, rowsB, semB)
    sc_wait(rowsA, dstvA, semSA)

    @pl.when(more)
    def _():
      loads(i + 2, srcvA, dstvA, wvA)
      g_start(srcvA, rowsA, semA)

    scale(rowsB, wvB)
    sc_start(rowsB, dstvB, semSB)
    sc_wait(rowsB, dstvB, semSB)

    @pl.when(more)
    def _():
      loads(i + 3, srcvB, dstvB, wvB)

    return 0

  lax.fori_loop(0, (_NCH_B - 1) // 2, pair, 0)
  plsc.subcore_barrier()

  for j in range(-(-_ZCH_B // NS)):
    cid = s + NS * j

    @pl.when(cid < _ZCH_B)
    def _():
      sl = pl.ds(cid * _KB, _KB)
      pltpu.sync_copy(acc.at[sl], rowsA)

      @pl.when(c == 0)
      def _():
        pltpu.sync_copy(rowsA, out0.at[sl])

      @pl.when(c == 1)
      def _():
        pltpu.sync_copy(rowsA, out1.at[sl])


@jax.jit
def _sc_b(h1a, h1b, ei, w):
  return pl.kernel(
      _scb_body,
      out_type=(
          jax.ShapeDtypeStruct((N_NODES, HALF), jnp.float32),
          jax.ShapeDtypeStruct((N_NODES, HALF), jnp.float32),
      ),
      mesh=_MESH,
      compiler_params=pltpu.CompilerParams(use_tc_tiling_on_sc=False),
      scratch_types=[
          pltpu.VMEM_SHARED((N_NODES, HALF), jnp.float32),
          pltpu.VMEM((_KB,), jnp.int32),
          pltpu.VMEM((_KB,), jnp.int32),
          pltpu.VMEM((_KB,), jnp.float32),
          pltpu.VMEM((_KB, HALF), jnp.float32),
          pltpu.VMEM((_KB,), jnp.int32),
          pltpu.VMEM((_KB,), jnp.int32),
          pltpu.VMEM((_KB,), jnp.float32),
          pltpu.VMEM((_KB, HALF), jnp.float32),
          pltpu.SemaphoreType.DMA,
          pltpu.SemaphoreType.DMA,
          pltpu.SemaphoreType.DMA,
          pltpu.SemaphoreType.DMA,
          pltpu.SemaphoreType.DMA,
      ],
  )(h1a, h1b, ei, w)


_KC = 2000
_EPW_C = N_EDGES // (NC * NS)
_NCH_C = _EPW_C // _KC
_NCH_N = N_NODES // _KC
_ACC_SZ = L * N_GRAPHS


def _scc_body(z0_hbm, z1_hbm, r0_hbm, r1_hbm, batch_hbm, ei_hbm,
              w_hbm, out,
              srcvA, dstvA, wvA, g0A, g1A, bdgA,
              srcvB, dstvB, wvB, g0B, g1B, bdgB,
              a0, a1, a2, a3, a4, obuf, semGA, semGB, semL):
  c = lax.axis_index("c")
  s = lax.axis_index("s")
  wid = c * NS + s
  lane128 = lax.iota(jnp.int32, L) * N_GRAPHS
  ones = jnp.full((L,), 1.0, jnp.float32)

  for a in (a0, a1, a2, a3, a4):
    _zero_vmem(a, _ACC_SZ)

  def eloads(i, srcv, dstv, wv):
    base = wid * _EPW_C + i * _KC
    d1 = pltpu.async_copy(ei_hbm.at[0, pl.ds(base, _KC)], srcv, semL)
    d2 = pltpu.async_copy(ei_hbm.at[1, pl.ds(base, _KC)], dstv, semL)
    d3 = pltpu.async_copy(w_hbm.at[pl.ds(base, _KC)], wv, semL)
    d1.wait(); d2.wait(); d3.wait()

  def g_start3(srcv, dstv, g0, g1, bdg, sem):
    pltpu.async_copy(z0_hbm.at[srcv], g0, sem)
    pltpu.async_copy(z1_hbm.at[srcv], g1, sem)
    pltpu.async_copy(batch_hbm.at[dstv], bdg, sem)

  def g_wait3(srcv, dstv, g0, g1, bdg, sem):
    pltpu.make_async_copy(z0_hbm.at[srcv], g0, sem).wait()
    pltpu.make_async_copy(z1_hbm.at[srcv], g1, sem).wait()
    pltpu.make_async_copy(batch_hbm.at[dstv], bdg, sem).wait()

  def accum(g0, g1, bdg, wv):
    def vec(j, _):
      sl = pl.ds(j * L, L)
      idx = bdg[sl] + lane128
      w16 = wv[sl]
      plsc.addupdate_scatter(a0, [idx], g0[sl] * w16)
      plsc.addupdate_scatter(a1, [idx], g1[sl] * w16)
      return 0

    lax.fori_loop(0, _KC // L, vec, 0)

  eloads(0, srcvA, dstvA, wvA)
  g_start3(srcvA, dstvA, g0A, g1A, bdgA, semGA)
  g_wait3(srcvA, dstvA, g0A, g1A, bdgA, semGA)
  accum(g0A, g1A, bdgA, wvA)

  eloads(1, srcvA, dstvA, wvA)
  g_start3(srcvA, dstvA, g0A, g1A, bdgA, semGA)
  eloads(2, srcvB, dstvB, wvB)

  def pair(i2, _):
    i = 1 + 2 * i2
    more = i2 < (_NCH_C - 1) // 2 - 1
    g_wait3(srcvA, dstvA, g0A, g1A, bdgA, semGA)
    g_start3(srcvB, dstvB, g0B, g1B, bdgB, semGB)
    accum(g0A, g1A, bdgA, wvA)

    @pl.when(more)
    def _():
      eloads(i + 2, srcvA, dstvA, wvA)
      g_start3(srcvA, dstvA, g0A, g1A, bdgA, semGA)

    g_wait3(srcvB, dstvB, g0B, g1B, bdgB, semGB)
    accum(g0B, g1B, bdgB, wvB)

    @pl.when(more)
    def _():
      eloads(i + 3, srcvB, dstvB, wvB)

    return 0

  lax.fori_loop(0, (_NCH_C - 1) // 2, pair, 0)

  for j in range(-(-_NCH_N // (NC * NS))):
    cid = wid + NC * NS * j

    @pl.when(cid < _NCH_N)
    def _():
      sl_h = pl.ds(cid * _KC, _KC)
      d1 = pltpu.async_copy(batch_hbm.at[sl_h], bdgA, semL)
      d2 = pltpu.async_copy(r0_hbm.at[sl_h], g0A, semL)
      d3 = pltpu.async_copy(r1_hbm.at[sl_h], g1A, semL)
      d1.wait(); d2.wait(); d3.wait()

      def vec(j2, _):
        sl = pl.ds(j2 * L, L)
        idx = bdgA[sl] + lane128
        plsc.addupdate_scatter(a2, [idx], g0A[sl])
        plsc.addupdate_scatter(a3, [idx], g1A[sl])
        plsc.addupdate_scatter(a4, [idx], ones)
        return 0

      lax.fori_loop(0, _KC // L, vec, 0)

  for p, a in enumerate((a0, a1, a2, a3, a4)):
    for j in range(N_GRAPHS // L):
      v = jnp.zeros((L,), jnp.float32)
      for lane in range(L):
        v = v + a[pl.ds(lane * N_GRAPHS + j * L, L)]
      obuf[p, pl.ds(j * L, L)] = v

  pltpu.sync_copy(obuf, out.at[wid])


@jax.jit
def _sc_c(z0, z1, r0, r1, batch, ei, w):
  return pl.kernel(
      _scc_body,
      out_type=jax.ShapeDtypeStruct((NC * NS, 5, N_GRAPHS), jnp.float32),
      mesh=_MESH,
      compiler_params=pltpu.CompilerParams(
          needs_layout_passes=False, use_tc_tiling_on_sc=False),
      scratch_types=[
          pltpu.VMEM((_KC,), jnp.int32),
          pltpu.VMEM((_KC,), jnp.int32),
          pltpu.VMEM((_KC,), jnp.float32),
          pltpu.VMEM((_KC,), jnp.float32),
          pltpu.VMEM((_KC,), jnp.float32),
          pltpu.VMEM((_KC,), jnp.int32),
          pltpu.VMEM((_KC,), jnp.int32),
          pltpu.VMEM((_KC,), jnp.int32),
          pltpu.VMEM((_KC,), jnp.float32),
          pltpu.VMEM((_KC,), jnp.float32),
          pltpu.VMEM((_KC,), jnp.float32),
          pltpu.VMEM((_KC,), jnp.int32),
          pltpu.VMEM((_ACC_SZ,), jnp.float32),
          pltpu.VMEM((_ACC_SZ,), jnp.float32),
          pltpu.VMEM((_ACC_SZ,), jnp.float32),
          pltpu.VMEM((_ACC_SZ,), jnp.float32),
          pltpu.VMEM((_ACC_SZ,), jnp.float32),
          pltpu.VMEM((5, N_GRAPHS), jnp.float32),
          pltpu.SemaphoreType.DMA,
          pltpu.SemaphoreType.DMA,
          pltpu.SemaphoreType.DMA,
      ],
  )(z0, z1, r0, r1, batch, ei, w)


_BT = 6144
_GT = -(-N_NODES // _BT)
_NPAD = _BT * _GT


def _tc1_body(a0_ref, a1_ref, x_ref, wrel_ref, wroot_ref, b1_ref, ha_ref,
              hb_ref):
  a = a0_ref[...] + a1_ref[...]
  xv = x_ref[...]
  pre = (a[:, None] * wrel_ref[0, :][None, :]
         + xv[:, None] * wroot_ref[0, :][None, :] + b1_ref[0, :][None, :])
  h = jnp.maximum(pre, 0.0)
  ha_ref[...] = h[:, :HALF]
  hb_ref[...] = h[:, HALF:]


@jax.jit
def _tc_1(a0, a1, x1d, W1_rel, W1_root, b1):
  return pl.pallas_call(
      _tc1_body,
      grid=(_GT,),
      in_specs=[
          pl.BlockSpec((_BT,), lambda i: (i,)),
          pl.BlockSpec((_BT,), lambda i: (i,)),
          pl.BlockSpec((_BT,), lambda i: (i,)),
          pl.BlockSpec((1, HIDDEN), lambda i: (0, 0)),
          pl.BlockSpec((1, HIDDEN), lambda i: (0, 0)),
          pl.BlockSpec((1, HIDDEN), lambda i: (0, 0)),
      ],
      out_specs=[
          pl.BlockSpec((_BT, HALF), lambda i: (i, 0)),
          pl.BlockSpec((_BT, HALF), lambda i: (i, 0)),
      ],
      out_shape=[
          jax.ShapeDtypeStruct((N_NODES, HALF), jnp.float32),
          jax.ShapeDtypeStruct((N_NODES, HALF), jnp.float32),
      ],
  )(a0, a1, x1d, W1_rel, W1_root, b1)


def _tc2_body(g0_ref, g1_ref, ha_ref, hb_ref, w2rel_ref, w2root_ref, b2_ref,
              w3rel_ref, w3root_ref, linw_ref, z0_ref, z1_ref, r0_ref, r1_ref):
  dot = lambda a, b: jnp.dot(a, b, preferred_element_type=jnp.float32)
  pre = (dot(g0_ref[...], w2rel_ref[:HALF, :])
         + dot(g1_ref[...], w2rel_ref[HALF:, :])
         + dot(ha_ref[...], w2root_ref[:HALF, :])
         + dot(hb_ref[...], w2root_ref[HALF:, :])
         + b2_ref[0, :][None, :])
  h2 = jnp.maximum(pre, 0.0)
  wz = dot(w3rel_ref[...], linw_ref[...])
  wr = dot(w3root_ref[...], linw_ref[...])
  z = dot(h2, wz)
  r = dot(h2, wr)
  z0_ref[...] = z[:, 0]
  z1_ref[...] = z[:, 1]
  r0_ref[...] = r[:, 0]
  r1_ref[...] = r[:, 1]


@jax.jit
def _tc_2(g0, g1, ha, hb, W2_rel, W2_root, b2, W3_rel, W3_root, lin_W):
  full = lambda r, c: pl.BlockSpec((r, c), lambda i: (0, 0))
  return pl.pallas_call(
      _tc2_body,
      grid=(_GT,),
      in_specs=[
          pl.BlockSpec((_BT, HALF), lambda i: (i, 0)),
          pl.BlockSpec((_BT, HALF), lambda i: (i, 0)),
          pl.BlockSpec((_BT, HALF), lambda i: (i, 0)),
          pl.BlockSpec((_BT, HALF), lambda i: (i, 0)),
          full(HIDDEN, HIDDEN),
          full(HIDDEN, HIDDEN),
          full(1, HIDDEN),
          full(HIDDEN, HIDDEN),
          full(HIDDEN, HIDDEN),
          full(HIDDEN, 2),
      ],
      out_specs=[pl.BlockSpec((_BT,), lambda i: (i,))] * 4,
      out_shape=[jax.ShapeDtypeStruct((_NPAD,), jnp.float32)] * 4,
  )(g0, g1, ha, hb, W2_rel, W2_root, b2, W3_rel, W3_root, lin_W)


def _tc3_body(p_ref, b3_ref, linw_ref, linb_ref, out_ref):
  sums = jnp.sum(p_ref[...], axis=0)
  e0, e1, r0, r1, n = (sums[0], sums[1], sums[2], sums[3], sums[4])
  cnt = jnp.maximum(n, 1.0)
  base = jnp.dot(b3_ref[...], linw_ref[...],
                 preferred_element_type=jnp.float32)
  nz = (n > 0.0).astype(jnp.float32)
  col0 = (e0 + r0) / cnt + nz * base[0, 0] + linb_ref[0, 0]
  col1 = (e1 + r1) / cnt + nz * base[0, 1] + linb_ref[0, 1]
  out_ref[...] = jnp.stack([col0, col1], axis=1)


@jax.jit
def _tc_3(partials, b3, lin_W, lin_b):
  return pl.pallas_call(
      _tc3_body,
      in_specs=[
          pl.BlockSpec((NC * NS, 5, N_GRAPHS), lambda: (0, 0, 0)),
          pl.BlockSpec((1, HIDDEN), lambda: (0, 0)),
          pl.BlockSpec((HIDDEN, 2), lambda: (0, 0)),
          pl.BlockSpec((1, 2), lambda: (0, 0)),
      ],
      out_specs=pl.BlockSpec((N_GRAPHS, 2), lambda: (0, 0)),
      out_shape=jax.ShapeDtypeStruct((N_GRAPHS, 2), jnp.float32),
  )(partials, b3, lin_W, lin_b)


def kernel(x, edge_index, batch, edge_weight, W1_rel, b1, W1_root, W2_rel, b2,
           W2_root, W3_rel, b3, W3_root, lin_W, lin_b):
  x1d = x[:, 0]

  ap0, ap1 = _sc_a(x1d, edge_index, edge_weight)
  ha, hb = _tc_1(ap0, ap1, x1d, W1_rel, W1_root, b1.reshape(1, HIDDEN))
  g0, g1 = _sc_b(ha, hb, edge_index, edge_weight)
  z0, z1, r0, r1 = _tc_2(g0, g1, ha, hb, W2_rel, W2_root,
                         b2.reshape(1, HIDDEN), W3_rel, W3_root, lin_W)
  partials = _sc_c(z0, z1, r0, r1, batch, edge_index, edge_weight)
  return _tc_3(partials, b3.reshape(1, HIDDEN), lin_W, lin_b.reshape(1, 2))
